# baseline (speedup 1.0000x reference)
import functools

import jax
import jax.numpy as jnp
from jax import lax
from jax.experimental import pallas as pl
from jax.experimental.pallas import tpu as pltpu
from jax.experimental.pallas import tpu_sc as plsc

_NC = 2
_NS = 16
_NW = _NC * _NS
_CK = 80


_NP = 10112
_NPS = _NP // _NS


def _attn_body(nh_hbm, eh_hbm, src_hbm, dst_hbm,
               attn_hbm, maxp_hbm, denp_hbm,
               si_v, di_v, s_v, d_v, e_v, attn_v, di_all, m_v, den_v,
               sem):
    c = lax.axis_index("c")
    s = lax.axis_index("s")
    wid = s * _NC + c
    E = eh_hbm.shape[0]
    ew = E // _NW
    base = wid * ew
    neg = jnp.full((16,), -1e30, jnp.float32)
    zer = jnp.zeros((16,), jnp.float32)

    def init(j, carry):
        sl = pl.ds(j * 16, 16)
        m_v[sl] = neg
        den_v[sl] = zer
        return carry

    lax.fori_loop(0, _NP // 16, init, 0)
    pltpu.sync_copy(dst_hbm.at[pl.ds(base, ew)], di_all)
    iota = lax.iota(jnp.int32, 16)

    def chunk(i, carry):
        b = base + i * _CK
        pltpu.sync_copy(src_hbm.at[pl.ds(b, _CK)], si_v)
        pltpu.sync_copy(dst_hbm.at[pl.ds(b, _CK)], di_v)
        pltpu.sync_copy(eh_hbm.at[pl.ds(b, _CK)], e_v)
        pltpu.async_copy(nh_hbm.at[si_v], s_v, sem).wait()
        pltpu.async_copy(nh_hbm.at[di_v], d_v, sem).wait()
        for g in range(_CK // 16):
            ridx = g * 16 + iota

            def cloop(cc, acc):
                cidx = iota * 0 + cc
                sv = plsc.load_gather(s_v, [ridx, cidx])
                ev = plsc.load_gather(e_v, [ridx, cidx])
                dv = plsc.load_gather(d_v, [ridx, cidx])
                return acc + (sv + ev) * dv

            acc = lax.fori_loop(0, 128, cloop, jnp.zeros((16,), jnp.float32))
            attn_v[pl.ds(i * _CK + g * 16, 16)] = acc
        return carry

    lax.fori_loop(0, ew // _CK, chunk, 0)

    def mloop(j, carry):
        sl = pl.ds(j * 16, 16)
        dd16 = di_all[sl]
        av = attn_v[sl]

        def mcond(pending):
            return jnp.any(pending)

        def mbody(pending):
            cur = plsc.load_gather(m_v, [dd16])
            plsc.store_scatter(m_v, [dd16], jnp.maximum(cur, av),
                               mask=pending)
            cur2 = plsc.load_gather(m_v, [dd16])
            return pending & (cur2 < av)

        lax.while_loop(mcond, mbody, iota < 16)
        return carry

    lax.fori_loop(0, ew // 16, mloop, 0)

    def eloop(j, carry):
        sl = pl.ds(j * 16, 16)
        dd16 = di_all[sl]
        mv = plsc.load_gather(m_v, [dd16])
        ex = jnp.exp(attn_v[sl] - mv)
        plsc.addupdate_scatter(den_v, [dd16], ex)
        return carry

    lax.fori_loop(0, ew // 16, eloop, 0)
    pltpu.sync_copy(attn_v, attn_hbm.at[pl.ds(base, ew)])
    pltpu.sync_copy(m_v, maxp_hbm.at[pl.ds(wid * _NP, _NP)])
    pltpu.sync_copy(den_v, denp_hbm.at[pl.ds(wid * _NP, _NP)])


def _attn_sc(nh, eh, src, dst):
    E = eh.shape[0]
    ew = E // _NW
    mesh = plsc.VectorSubcoreMesh(core_axis_name="c", subcore_axis_name="s")
    f = pl.kernel(
        _attn_body,
        out_type=[
            jax.ShapeDtypeStruct((E,), jnp.float32),
            jax.ShapeDtypeStruct((_NW * _NP,), jnp.float32),
            jax.ShapeDtypeStruct((_NW * _NP,), jnp.float32),
        ],
        mesh=mesh,
        scratch_types=[
            pltpu.VMEM((_CK,), jnp.int32),
            pltpu.VMEM((_CK,), jnp.int32),
            pltpu.VMEM((_CK, 128), jnp.float32),
            pltpu.VMEM((_CK, 128), jnp.float32),
            pltpu.VMEM((_CK, 128), jnp.float32),
            pltpu.VMEM((ew,), jnp.float32),
            pltpu.VMEM((ew,), jnp.int32),
            pltpu.VMEM((_NP,), jnp.float32),
            pltpu.VMEM((_NP,), jnp.float32),
            pltpu.SemaphoreType.DMA,
        ],
        compiler_params=pltpu.CompilerParams(needs_layout_passes=False),
    )
    return f(nh, eh, src, dst)


_NHALF = 5056
_NPH = 5120


def _scatter_body(nh_hbm, attn_hbm, src_hbm, dst_hbm, maxp_hbm, denp_hbm,
                  nzp_hbm, M_hbm, D_hbm,
                  si_v, di_v, di2_v, at_v, a_v, s_v, zbuf, mw_buf, dw_buf,
                  M_v, D_v, nz_sh, sem):
    c = lax.axis_index("c")
    s = lax.axis_index("s")
    wid = s * _NC + c
    E = attn_hbm.shape[0]
    ew = E // _NW
    base = wid * ew
    ns_base = s * _NPS
    iota = lax.iota(jnp.int32, 16)
    zer16 = jnp.zeros((16,), jnp.float32)
    nq = (_NPS + 15) // 16

    def minit(q, carry):
        lidx = q * 16 + iota
        msk = lidx < _NPS
        plsc.store_scatter(M_v, [ns_base + lidx],
                           jnp.full((16,), -1e30, jnp.float32), mask=msk)
        plsc.store_scatter(D_v, [ns_base + lidx], zer16, mask=msk)
        return carry

    lax.fori_loop(0, nq, minit, 0)

    def wmax(w, carry):
        pltpu.sync_copy(maxp_hbm.at[pl.ds(w * _NP + ns_base, _NPS)],
                        mw_buf.at[pl.ds(0, _NPS)])

        def q1(q, carry2):
            lidx = q * 16 + iota
            msk = lidx < _NPS
            idxs = ns_base + lidx
            cur = plsc.load_gather(M_v, [idxs], mask=msk)
            mw = plsc.load_gather(mw_buf, [lidx], mask=msk)
            plsc.store_scatter(M_v, [idxs], jnp.maximum(cur, mw), mask=msk)
            return carry2

        lax.fori_loop(0, nq, q1, 0)
        return carry

    lax.fori_loop(0, _NW, wmax, 0)

    def wden(w, carry):
        pltpu.sync_copy(maxp_hbm.at[pl.ds(w * _NP + ns_base, _NPS)],
                        mw_buf.at[pl.ds(0, _NPS)])
        pltpu.sync_copy(denp_hbm.at[pl.ds(w * _NP + ns_base, _NPS)],
                        dw_buf.at[pl.ds(0, _NPS)])

        def q2(q, carry2):
            lidx = q * 16 + iota
            msk = lidx < _NPS
            idxs = ns_base + lidx
            cur = plsc.load_gather(D_v, [idxs], mask=msk)
            mw = plsc.load_gather(mw_buf, [lidx], mask=msk)
            dw = plsc.load_gather(dw_buf, [lidx], mask=msk)
            mfin = plsc.load_gather(M_v, [idxs], mask=msk)
            plsc.store_scatter(D_v, [idxs], cur + dw * jnp.exp(mw - mfin),
                               mask=msk)
            return carry2

        lax.fori_loop(0, nq, q2, 0)
        return carry

    lax.fori_loop(0, _NW, wden, 0)
    pltpu.sync_copy(M_v.at[pl.ds(ns_base, _NPS)], M_hbm.at[pl.ds(ns_base, _NPS)])
    pltpu.sync_copy(D_v.at[pl.ds(ns_base, _NPS)], D_hbm.at[pl.ds(ns_base, _NPS)])
    plsc.subcore_barrier()
    pltpu.sync_copy(M_hbm, M_v.at[pl.ds(0, _NP)])
    pltpu.sync_copy(D_hbm, D_v.at[pl.ds(0, _NP)])

    def zero_zbuf():
        msk8 = iota < 8

        def zcol(cc, carry2):
            cidx = iota * 0 + cc
            plsc.store_scatter(zbuf, [iota, cidx], zer16, mask=msk8)
            return carry2

        lax.fori_loop(0, 128, zcol, 0)

    ws_base = s * (_NPH // _NS)
    for p in range(2):
        zero_zbuf()
        for q in range(_NPH // _NS // 8):
            pltpu.sync_copy(zbuf, nz_sh.at[pl.ds(ws_base + q * 8, 8)])
        plsc.subcore_barrier()

        def chunk(i, carry):
            b = base + i * _CK
            pltpu.sync_copy(src_hbm.at[pl.ds(b, _CK)], si_v)
            pltpu.sync_copy(dst_hbm.at[pl.ds(b, _CK)], di_v)
            pltpu.sync_copy(attn_hbm.at[pl.ds(b, _CK)], at_v)
            pltpu.async_copy(nh_hbm.at[si_v], s_v, sem).wait()
            for g in range(_CK // 16):
                sl = pl.ds(g * 16, 16)
                ridx = g * 16 + iota
                dl = di_v[sl]
                mv = plsc.load_gather(M_v, [dl])
                dv = plsc.load_gather(D_v, [dl])
                a16 = jnp.exp(at_v[sl] - mv) / dv
                loc = dl - p * _NHALF
                ok = (loc >= 0) & (loc < _NHALF)
                di2_v[sl] = jnp.where(ok, loc, _NHALF)

                def ccol(cc, carry2):
                    cidx = iota * 0 + cc
                    sv = plsc.load_gather(s_v, [ridx, cidx])
                    plsc.store_scatter(s_v, [ridx, cidx], sv * a16)
                    return carry2

                lax.fori_loop(0, 128, ccol, 0)

            pltpu.sync_copy(s_v, nz_sh.at[di2_v], add=True)
            return carry

        lax.fori_loop(0, ew // _CK, chunk, 0)
        plsc.subcore_barrier()
        for q in range(_NPH // _NS // 8):
            pltpu.sync_copy(nz_sh.at[pl.ds(ws_base + q * 8, 8)], zbuf)
            pltpu.sync_copy(zbuf,
                            nzp_hbm.at[c, p, pl.ds(ws_base + q * 8, 8)])


def _scatter_sc(nh, attn, src, dst, maxp, denp):
    E = attn.shape[0]
    mesh = plsc.VectorSubcoreMesh(core_axis_name="c", subcore_axis_name="s")
    f = pl.kernel(
        _scatter_body,
        out_type=[
            jax.ShapeDtypeStruct((_NC, 2, _NPH, 128), jnp.float32),
            jax.ShapeDtypeStruct((_NP,), jnp.float32),
            jax.ShapeDtypeStruct((_NP,), jnp.float32),
        ],
        mesh=mesh,
        scratch_types=[
            pltpu.VMEM((_CK,), jnp.int32),
            pltpu.VMEM((_CK,), jnp.int32),
            pltpu.VMEM((_CK,), jnp.int32),
            pltpu.VMEM((_CK,), jnp.float32),
            pltpu.VMEM((_CK,), jnp.float32),
            pltpu.VMEM((_CK, 128), jnp.float32),
            pltpu.VMEM((8, 128), jnp.float32),
            pltpu.VMEM((640,), jnp.float32),
            pltpu.VMEM((640,), jnp.float32),
            pltpu.VMEM((_NP + 32,), jnp.float32),
            pltpu.VMEM((_NP + 32,), jnp.float32),
            pltpu.VMEM_SHARED((_NPH, 128), jnp.float32),
            pltpu.SemaphoreType.DMA,
        ],
        compiler_params=pltpu.CompilerParams(needs_layout_passes=False),
    )
    return f(nh, attn, src, dst, maxp, denp)[0]


def _epre_body(epsp_hbm, eh_hbm, nz_hbm, src_hbm, dst_hbm, out_hbm,
               epsp_v, eh_v, s_v, d_v, si_v, di_v, sem):
    c = lax.axis_index("c")
    s = lax.axis_index("s")
    wid = s * _NC + c
    E = eh_hbm.shape[0]
    ew = E // _NW
    base = wid * ew
    pltpu.sync_copy(epsp_hbm, epsp_v)
    eps_sl = [epsp_v[pl.ds(t * 16, 16)] for t in range(8)]

    def chunk(i, carry):
        b = base + i * _CK
        pltpu.sync_copy(src_hbm.at[pl.ds(b, _CK)], si_v)
        pltpu.sync_copy(dst_hbm.at[pl.ds(b, _CK)], di_v)
        pltpu.sync_copy(eh_hbm.at[pl.ds(b, _CK)], eh_v)
        pltpu.async_copy(nz_hbm.at[si_v], s_v, sem).wait()
        pltpu.async_copy(nz_hbm.at[di_v], d_v, sem).wait()

        def row(j, carry2):
            for t in range(8):
                sl = pl.ds(t * 16, 16)
                eh_v[j, sl] = (eh_v[j, sl] * eps_sl[t]
                               + s_v[j, sl] - d_v[j, sl])
            return carry2

        lax.fori_loop(0, _CK, row, 0)
        pltpu.sync_copy(eh_v, out_hbm.at[pl.ds(b, _CK)])
        return carry

    lax.fori_loop(0, ew // _CK, chunk, 0)


def _epre_sc(epsp, eh, nz, src, dst):
    E = eh.shape[0]
    mesh = plsc.VectorSubcoreMesh(core_axis_name="c", subcore_axis_name="s")
    f = pl.kernel(
        _epre_body,
        out_type=jax.ShapeDtypeStruct((E, 128), jnp.float32),
        mesh=mesh,
        scratch_types=[
            pltpu.VMEM((128,), jnp.float32),
            pltpu.VMEM((_CK, 128), jnp.float32),
            pltpu.VMEM((_CK, 128), jnp.float32),
            pltpu.VMEM((_CK, 128), jnp.float32),
            pltpu.VMEM((_CK,), jnp.int32),
            pltpu.VMEM((_CK,), jnp.int32),
            pltpu.SemaphoreType.DMA,
        ],
    )
    return f(epsp, eh, nz, src, dst)


def _mlp_stats_body(x_ref, w1_ref, b1_ref, w2_ref, b2_ref,
                    y_ref, s1_ref, s2_ref):
    i = pl.program_id(0)
    x = x_ref[...]
    h = jnp.maximum(
        jnp.dot(x, w1_ref[...], preferred_element_type=jnp.float32)
        + b1_ref[...], 0.0)
    y = (jnp.dot(h, w2_ref[...], preferred_element_type=jnp.float32)
         + b2_ref[...])
    y_ref[...] = y

    @pl.when(i == 0)
    def _init():
        s1_ref[...] = jnp.zeros_like(s1_ref)
        s2_ref[...] = jnp.zeros_like(s2_ref)

    s1_ref[...] += jnp.sum(y, axis=0, keepdims=True)
    s2_ref[...] += jnp.sum(y * y, axis=0, keepdims=True)


def _mlp_stats(x, w1, b1, w2, b2, block):
    n = x.shape[0]
    grid = n // block
    y, s1, s2 = pl.pallas_call(
        _mlp_stats_body,
        grid=(grid,),
        in_specs=[
            pl.BlockSpec((block, 128), lambda i: (i, 0)),
            pl.BlockSpec((128, 128), lambda i: (0, 0)),
            pl.BlockSpec((1, 128), lambda i: (0, 0)),
            pl.BlockSpec((128, 128), lambda i: (0, 0)),
            pl.BlockSpec((1, 128), lambda i: (0, 0)),
        ],
        out_specs=[
            pl.BlockSpec((block, 128), lambda i: (i, 0)),
            pl.BlockSpec((1, 128), lambda i: (0, 0)),
            pl.BlockSpec((1, 128), lambda i: (0, 0)),
        ],
        out_shape=[
            jax.ShapeDtypeStruct((n, 128), jnp.float32),
            jax.ShapeDtypeStruct((1, 128), jnp.float32),
            jax.ShapeDtypeStruct((1, 128), jnp.float32),
        ],
    )(x, w1, b1.reshape(1, 128), w2, b2.reshape(1, 128))
    return y, s1, s2


def _bn_body(y_ref, mu_ref, rstd_ref, gamma_ref, beta_ref, o_ref):
    o_ref[...] = ((y_ref[...] - mu_ref[...]) * rstd_ref[...]
                  * gamma_ref[...] + beta_ref[...])


def _bn_apply(y, mu, rstd, gamma, beta, block):
    n = y.shape[0]
    return pl.pallas_call(
        _bn_body,
        grid=(n // block,),
        in_specs=[
            pl.BlockSpec((block, 128), lambda i: (i, 0)),
            pl.BlockSpec((1, 128), lambda i: (0, 0)),
            pl.BlockSpec((1, 128), lambda i: (0, 0)),
            pl.BlockSpec((1, 128), lambda i: (0, 0)),
            pl.BlockSpec((1, 128), lambda i: (0, 0)),
        ],
        out_specs=pl.BlockSpec((block, 128), lambda i: (i, 0)),
        out_shape=jax.ShapeDtypeStruct((n, 128), jnp.float32),
    )(y, mu, rstd, gamma.reshape(1, 128), beta.reshape(1, 128))


def _mlp_bn(x, w1, b1, w2, b2, gamma, beta, block):
    n = x.shape[0]
    y, s1, s2 = _mlp_stats(x, w1, b1, w2, b2, block)
    mu = s1 / n
    var = s2 / n - mu * mu
    rstd = jax.lax.rsqrt(var + 1e-5)
    return _bn_apply(y, mu, rstd, gamma, beta, block)


def kernel(nh, eh, edge_index,
           nf_W1, nf_b1, nf_W2, nf_b2, nf_eps, nf_gamma, nf_beta,
           ef_W1, ef_b1, ef_W2, ef_b2, ef_eps, ef_gamma, ef_beta):
    N = nh.shape[0]
    src, dst = edge_index[0], edge_index[1]
    attn, maxp, denp = _attn_sc(nh, eh, src, dst)
    nzp = _scatter_sc(nh, attn, src, dst, maxp, denp)
    nz = jnp.concatenate([(nzp[0, 0] + nzp[1, 0])[:_NHALF],
                          (nzp[0, 1] + nzp[1, 1])[:N - _NHALF]], axis=0)
    n_pre = (1.0 + nf_eps) * nh + nz
    e_pre = _epre_sc(1.0 + ef_eps, eh, nz, src, dst)
    n_h = _mlp_bn(n_pre, nf_W1, nf_b1, nf_W2, nf_b2, nf_gamma, nf_beta, 1000)
    e_h = _mlp_bn(e_pre, ef_W1, ef_b1, ef_W2, ef_b2, ef_gamma, ef_beta, 1000)
    return (n_h, e_h)


# speedup vs baseline: 1.9723x; 1.9723x over previous
import functools

import jax
import jax.numpy as jnp
from jax import lax
from jax.experimental import pallas as pl
from jax.experimental.pallas import tpu as pltpu
from jax.experimental.pallas import tpu_sc as plsc

_NC = 2
_NS = 16
_NW = _NC * _NS
_CK = 80


_NP = 10112
_NPS = _NP // _NS


def _attn_body(nh_hbm, eh_hbm, src_hbm, dst_hbm,
               attn_hbm, maxp_hbm, denp_hbm,
               si_v, di_v, s_v, d_v, e_v, attn_v, di_all, m_v, den_v,
               sem):
    c = lax.axis_index("c")
    s = lax.axis_index("s")
    wid = s * _NC + c
    E = eh_hbm.shape[0]
    ew = E // _NW
    base = wid * ew
    neg = jnp.full((16,), -1e30, jnp.float32)
    zer = jnp.zeros((16,), jnp.float32)

    def init(j, carry):
        sl = pl.ds(j * 16, 16)
        m_v[sl] = neg
        den_v[sl] = zer
        return carry

    lax.fori_loop(0, _NP // 16, init, 0)
    pltpu.sync_copy(dst_hbm.at[pl.ds(base, ew)], di_all)
    iota = lax.iota(jnp.int32, 16)

    def chunk(i, carry):
        b = base + i * _CK
        pltpu.sync_copy(src_hbm.at[pl.ds(b, _CK)], si_v)
        pltpu.sync_copy(dst_hbm.at[pl.ds(b, _CK)], di_v)
        pltpu.sync_copy(eh_hbm.at[pl.ds(b, _CK)], e_v)
        pltpu.async_copy(nh_hbm.at[si_v], s_v, sem).wait()
        pltpu.async_copy(nh_hbm.at[di_v], d_v, sem).wait()
        for g in range(_CK // 16):
            ridx = g * 16 + iota

            def cloop(cc, acc):
                cidx = (iota + cc) & 127
                sv = plsc.load_gather(s_v, [ridx, cidx])
                ev = plsc.load_gather(e_v, [ridx, cidx])
                dv = plsc.load_gather(d_v, [ridx, cidx])
                return acc + (sv + ev) * dv

            acc = lax.fori_loop(0, 128, cloop, jnp.zeros((16,), jnp.float32))
            attn_v[pl.ds(i * _CK + g * 16, 16)] = acc
        return carry

    lax.fori_loop(0, ew // _CK, chunk, 0)

    def mloop(j, carry):
        sl = pl.ds(j * 16, 16)
        dd16 = di_all[sl]
        av = attn_v[sl]

        def mcond(pending):
            return jnp.any(pending)

        def mbody(pending):
            cur = plsc.load_gather(m_v, [dd16])
            plsc.store_scatter(m_v, [dd16], jnp.maximum(cur, av),
                               mask=pending)
            cur2 = plsc.load_gather(m_v, [dd16])
            return pending & (cur2 < av)

        lax.while_loop(mcond, mbody, iota < 16)
        return carry

    lax.fori_loop(0, ew // 16, mloop, 0)

    def eloop(j, carry):
        sl = pl.ds(j * 16, 16)
        dd16 = di_all[sl]
        mv = plsc.load_gather(m_v, [dd16])
        ex = jnp.exp(attn_v[sl] - mv)
        plsc.addupdate_scatter(den_v, [dd16], ex)
        return carry

    lax.fori_loop(0, ew // 16, eloop, 0)
    pltpu.sync_copy(attn_v, attn_hbm.at[pl.ds(base, ew)])
    pltpu.sync_copy(m_v, maxp_hbm.at[pl.ds(wid * _NP, _NP)])
    pltpu.sync_copy(den_v, denp_hbm.at[pl.ds(wid * _NP, _NP)])


def _attn_sc(nh, eh, src, dst):
    E = eh.shape[0]
    ew = E // _NW
    mesh = plsc.VectorSubcoreMesh(core_axis_name="c", subcore_axis_name="s")
    f = pl.kernel(
        _attn_body,
        out_type=[
            jax.ShapeDtypeStruct((E,), jnp.float32),
            jax.ShapeDtypeStruct((_NW * _NP,), jnp.float32),
            jax.ShapeDtypeStruct((_NW * _NP,), jnp.float32),
        ],
        mesh=mesh,
        scratch_types=[
            pltpu.VMEM((_CK,), jnp.int32),
            pltpu.VMEM((_CK,), jnp.int32),
            pltpu.VMEM((_CK, 128), jnp.float32),
            pltpu.VMEM((_CK, 128), jnp.float32),
            pltpu.VMEM((_CK, 128), jnp.float32),
            pltpu.VMEM((ew,), jnp.float32),
            pltpu.VMEM((ew,), jnp.int32),
            pltpu.VMEM((_NP,), jnp.float32),
            pltpu.VMEM((_NP,), jnp.float32),
            pltpu.SemaphoreType.DMA,
        ],
        compiler_params=pltpu.CompilerParams(needs_layout_passes=False),
    )
    return f(nh, eh, src, dst)


_NHALF = 5056
_NPH = 5120


def _scatter_body(nh_hbm, attn_hbm, src_hbm, dst_hbm, maxp_hbm, denp_hbm,
                  nzp_hbm, M_hbm, D_hbm,
                  si_v, di_v, di2_v, at_v, a_v, s_v, zbuf, mw_buf, dw_buf,
                  M_v, D_v, nz_sh, sem):
    c = lax.axis_index("c")
    s = lax.axis_index("s")
    wid = s * _NC + c
    E = attn_hbm.shape[0]
    ew = E // _NW
    base = wid * ew
    ns_base = s * _NPS
    iota = lax.iota(jnp.int32, 16)
    zer16 = jnp.zeros((16,), jnp.float32)
    nq = (_NPS + 15) // 16

    def minit(q, carry):
        lidx = q * 16 + iota
        msk = lidx < _NPS
        plsc.store_scatter(M_v, [ns_base + lidx],
                           jnp.full((16,), -1e30, jnp.float32), mask=msk)
        plsc.store_scatter(D_v, [ns_base + lidx], zer16, mask=msk)
        return carry

    lax.fori_loop(0, nq, minit, 0)

    def wmax(w, carry):
        pltpu.sync_copy(maxp_hbm.at[pl.ds(w * _NP + ns_base, _NPS)],
                        mw_buf.at[pl.ds(0, _NPS)])

        def q1(q, carry2):
            lidx = q * 16 + iota
            msk = lidx < _NPS
            idxs = ns_base + lidx
            cur = plsc.load_gather(M_v, [idxs], mask=msk)
            mw = plsc.load_gather(mw_buf, [lidx], mask=msk)
            plsc.store_scatter(M_v, [idxs], jnp.maximum(cur, mw), mask=msk)
            return carry2

        lax.fori_loop(0, nq, q1, 0)
        return carry

    lax.fori_loop(0, _NW, wmax, 0)

    def wden(w, carry):
        pltpu.sync_copy(maxp_hbm.at[pl.ds(w * _NP + ns_base, _NPS)],
                        mw_buf.at[pl.ds(0, _NPS)])
        pltpu.sync_copy(denp_hbm.at[pl.ds(w * _NP + ns_base, _NPS)],
                        dw_buf.at[pl.ds(0, _NPS)])

        def q2(q, carry2):
            lidx = q * 16 + iota
            msk = lidx < _NPS
            idxs = ns_base + lidx
            cur = plsc.load_gather(D_v, [idxs], mask=msk)
            mw = plsc.load_gather(mw_buf, [lidx], mask=msk)
            dw = plsc.load_gather(dw_buf, [lidx], mask=msk)
            mfin = plsc.load_gather(M_v, [idxs], mask=msk)
            plsc.store_scatter(D_v, [idxs], cur + dw * jnp.exp(mw - mfin),
                               mask=msk)
            return carry2

        lax.fori_loop(0, nq, q2, 0)
        return carry

    lax.fori_loop(0, _NW, wden, 0)
    pltpu.sync_copy(M_v.at[pl.ds(ns_base, _NPS)], M_hbm.at[pl.ds(ns_base, _NPS)])
    pltpu.sync_copy(D_v.at[pl.ds(ns_base, _NPS)], D_hbm.at[pl.ds(ns_base, _NPS)])
    plsc.subcore_barrier()
    pltpu.sync_copy(M_hbm, M_v.at[pl.ds(0, _NP)])
    pltpu.sync_copy(D_hbm, D_v.at[pl.ds(0, _NP)])

    def zero_zbuf():
        msk8 = iota < 8

        def zcol(cc, carry2):
            cidx = iota * 0 + cc
            plsc.store_scatter(zbuf, [iota, cidx], zer16, mask=msk8)
            return carry2

        lax.fori_loop(0, 128, zcol, 0)

    ws_base = s * (_NPH // _NS)
    for p in range(2):
        zero_zbuf()
        for q in range(_NPH // _NS // 8):
            pltpu.sync_copy(zbuf, nz_sh.at[pl.ds(ws_base + q * 8, 8)])
        plsc.subcore_barrier()

        def chunk(i, carry):
            b = base + i * _CK
            pltpu.sync_copy(src_hbm.at[pl.ds(b, _CK)], si_v)
            pltpu.sync_copy(dst_hbm.at[pl.ds(b, _CK)], di_v)
            pltpu.sync_copy(attn_hbm.at[pl.ds(b, _CK)], at_v)
            pltpu.async_copy(nh_hbm.at[si_v], s_v, sem).wait()
            for g in range(_CK // 16):
                sl = pl.ds(g * 16, 16)
                ridx = g * 16 + iota
                dl = di_v[sl]
                mv = plsc.load_gather(M_v, [dl])
                dv = plsc.load_gather(D_v, [dl])
                a16 = jnp.exp(at_v[sl] - mv) / dv
                loc = dl - p * _NHALF
                ok = (loc >= 0) & (loc < _NHALF)
                di2_v[sl] = jnp.where(ok, loc, _NHALF)

                def ccol(cc, carry2):
                    cidx = (iota + cc) & 127
                    sv = plsc.load_gather(s_v, [ridx, cidx])
                    plsc.store_scatter(s_v, [ridx, cidx], sv * a16)
                    return carry2

                lax.fori_loop(0, 128, ccol, 0)

            pltpu.sync_copy(s_v, nz_sh.at[di2_v], add=True)
            return carry

        lax.fori_loop(0, ew // _CK, chunk, 0)
        plsc.subcore_barrier()
        for q in range(_NPH // _NS // 8):
            pltpu.sync_copy(nz_sh.at[pl.ds(ws_base + q * 8, 8)], zbuf)
            pltpu.sync_copy(zbuf,
                            nzp_hbm.at[c, p, pl.ds(ws_base + q * 8, 8)])


def _scatter_sc(nh, attn, src, dst, maxp, denp):
    E = attn.shape[0]
    mesh = plsc.VectorSubcoreMesh(core_axis_name="c", subcore_axis_name="s")
    f = pl.kernel(
        _scatter_body,
        out_type=[
            jax.ShapeDtypeStruct((_NC, 2, _NPH, 128), jnp.float32),
            jax.ShapeDtypeStruct((_NP,), jnp.float32),
            jax.ShapeDtypeStruct((_NP,), jnp.float32),
        ],
        mesh=mesh,
        scratch_types=[
            pltpu.VMEM((_CK,), jnp.int32),
            pltpu.VMEM((_CK,), jnp.int32),
            pltpu.VMEM((_CK,), jnp.int32),
            pltpu.VMEM((_CK,), jnp.float32),
            pltpu.VMEM((_CK,), jnp.float32),
            pltpu.VMEM((_CK, 128), jnp.float32),
            pltpu.VMEM((8, 128), jnp.float32),
            pltpu.VMEM((640,), jnp.float32),
            pltpu.VMEM((640,), jnp.float32),
            pltpu.VMEM((_NP + 32,), jnp.float32),
            pltpu.VMEM((_NP + 32,), jnp.float32),
            pltpu.VMEM_SHARED((_NPH, 128), jnp.float32),
            pltpu.SemaphoreType.DMA,
        ],
        compiler_params=pltpu.CompilerParams(needs_layout_passes=False),
    )
    return f(nh, attn, src, dst, maxp, denp)[0]


def _epre_body(epsp_hbm, eh_hbm, nz_hbm, src_hbm, dst_hbm, out_hbm,
               epsp_v, eh_v, s_v, d_v, si_v, di_v, sem):
    c = lax.axis_index("c")
    s = lax.axis_index("s")
    wid = s * _NC + c
    E = eh_hbm.shape[0]
    ew = E // _NW
    base = wid * ew
    pltpu.sync_copy(epsp_hbm, epsp_v)
    eps_sl = [epsp_v[pl.ds(t * 16, 16)] for t in range(8)]

    def chunk(i, carry):
        b = base + i * _CK
        pltpu.sync_copy(src_hbm.at[pl.ds(b, _CK)], si_v)
        pltpu.sync_copy(dst_hbm.at[pl.ds(b, _CK)], di_v)
        pltpu.sync_copy(eh_hbm.at[pl.ds(b, _CK)], eh_v)
        pltpu.async_copy(nz_hbm.at[si_v], s_v, sem).wait()
        pltpu.async_copy(nz_hbm.at[di_v], d_v, sem).wait()

        def row(j, carry2):
            for t in range(8):
                sl = pl.ds(t * 16, 16)
                eh_v[j, sl] = (eh_v[j, sl] * eps_sl[t]
                               + s_v[j, sl] - d_v[j, sl])
            return carry2

        lax.fori_loop(0, _CK, row, 0)
        pltpu.sync_copy(eh_v, out_hbm.at[pl.ds(b, _CK)])
        return carry

    lax.fori_loop(0, ew // _CK, chunk, 0)


def _epre_sc(epsp, eh, nz, src, dst):
    E = eh.shape[0]
    mesh = plsc.VectorSubcoreMesh(core_axis_name="c", subcore_axis_name="s")
    f = pl.kernel(
        _epre_body,
        out_type=jax.ShapeDtypeStruct((E, 128), jnp.float32),
        mesh=mesh,
        scratch_types=[
            pltpu.VMEM((128,), jnp.float32),
            pltpu.VMEM((_CK, 128), jnp.float32),
            pltpu.VMEM((_CK, 128), jnp.float32),
            pltpu.VMEM((_CK, 128), jnp.float32),
            pltpu.VMEM((_CK,), jnp.int32),
            pltpu.VMEM((_CK,), jnp.int32),
            pltpu.SemaphoreType.DMA,
        ],
    )
    return f(epsp, eh, nz, src, dst)


def _mlp_stats_body(x_ref, w1_ref, b1_ref, w2_ref, b2_ref,
                    y_ref, s1_ref, s2_ref):
    i = pl.program_id(0)
    x = x_ref[...]
    h = jnp.maximum(
        jnp.dot(x, w1_ref[...], preferred_element_type=jnp.float32)
        + b1_ref[...], 0.0)
    y = (jnp.dot(h, w2_ref[...], preferred_element_type=jnp.float32)
         + b2_ref[...])
    y_ref[...] = y

    @pl.when(i == 0)
    def _init():
        s1_ref[...] = jnp.zeros_like(s1_ref)
        s2_ref[...] = jnp.zeros_like(s2_ref)

    s1_ref[...] += jnp.sum(y, axis=0, keepdims=True)
    s2_ref[...] += jnp.sum(y * y, axis=0, keepdims=True)


def _mlp_stats(x, w1, b1, w2, b2, block):
    n = x.shape[0]
    grid = n // block
    y, s1, s2 = pl.pallas_call(
        _mlp_stats_body,
        grid=(grid,),
        in_specs=[
            pl.BlockSpec((block, 128), lambda i: (i, 0)),
            pl.BlockSpec((128, 128), lambda i: (0, 0)),
            pl.BlockSpec((1, 128), lambda i: (0, 0)),
            pl.BlockSpec((128, 128), lambda i: (0, 0)),
            pl.BlockSpec((1, 128), lambda i: (0, 0)),
        ],
        out_specs=[
            pl.BlockSpec((block, 128), lambda i: (i, 0)),
            pl.BlockSpec((1, 128), lambda i: (0, 0)),
            pl.BlockSpec((1, 128), lambda i: (0, 0)),
        ],
        out_shape=[
            jax.ShapeDtypeStruct((n, 128), jnp.float32),
            jax.ShapeDtypeStruct((1, 128), jnp.float32),
            jax.ShapeDtypeStruct((1, 128), jnp.float32),
        ],
    )(x, w1, b1.reshape(1, 128), w2, b2.reshape(1, 128))
    return y, s1, s2


def _bn_body(y_ref, mu_ref, rstd_ref, gamma_ref, beta_ref, o_ref):
    o_ref[...] = ((y_ref[...] - mu_ref[...]) * rstd_ref[...]
                  * gamma_ref[...] + beta_ref[...])


def _bn_apply(y, mu, rstd, gamma, beta, block):
    n = y.shape[0]
    return pl.pallas_call(
        _bn_body,
        grid=(n // block,),
        in_specs=[
            pl.BlockSpec((block, 128), lambda i: (i, 0)),
            pl.BlockSpec((1, 128), lambda i: (0, 0)),
            pl.BlockSpec((1, 128), lambda i: (0, 0)),
            pl.BlockSpec((1, 128), lambda i: (0, 0)),
            pl.BlockSpec((1, 128), lambda i: (0, 0)),
        ],
        out_specs=pl.BlockSpec((block, 128), lambda i: (i, 0)),
        out_shape=jax.ShapeDtypeStruct((n, 128), jnp.float32),
    )(y, mu, rstd, gamma.reshape(1, 128), beta.reshape(1, 128))


def _mlp_bn(x, w1, b1, w2, b2, gamma, beta, block):
    n = x.shape[0]
    y, s1, s2 = _mlp_stats(x, w1, b1, w2, b2, block)
    mu = s1 / n
    var = s2 / n - mu * mu
    rstd = jax.lax.rsqrt(var + 1e-5)
    return _bn_apply(y, mu, rstd, gamma, beta, block)


def kernel(nh, eh, edge_index,
           nf_W1, nf_b1, nf_W2, nf_b2, nf_eps, nf_gamma, nf_beta,
           ef_W1, ef_b1, ef_W2, ef_b2, ef_eps, ef_gamma, ef_beta):
    N = nh.shape[0]
    src, dst = edge_index[0], edge_index[1]
    attn, maxp, denp = _attn_sc(nh, eh, src, dst)
    nzp = _scatter_sc(nh, attn, src, dst, maxp, denp)
    nz = jnp.concatenate([(nzp[0, 0] + nzp[1, 0])[:_NHALF],
                          (nzp[0, 1] + nzp[1, 1])[:N - _NHALF]], axis=0)
    n_pre = (1.0 + nf_eps) * nh + nz
    e_pre = _epre_sc(1.0 + ef_eps, eh, nz, src, dst)
    n_h = _mlp_bn(n_pre, nf_W1, nf_b1, nf_W2, nf_b2, nf_gamma, nf_beta, 1000)
    e_h = _mlp_bn(e_pre, ef_W1, ef_b1, ef_W2, ef_b2, ef_gamma, ef_beta, 1000)
    return (n_h, e_h)


# speedup vs baseline: 2.2059x; 1.1184x over previous
import functools

import jax
import jax.numpy as jnp
from jax import lax
from jax.experimental import pallas as pl
from jax.experimental.pallas import tpu as pltpu
from jax.experimental.pallas import tpu_sc as plsc

_NC = 2
_NS = 16
_NW = _NC * _NS
_CK = 80


_NP = 10112
_NPS = _NP // _NS


def _attn_body(nh_hbm, eh_hbm, src_hbm, dst_hbm,
               attn_hbm, maxp_hbm, denp_hbm,
               si_v, di_v, s_v, d_v, e_v, attn_v, di_all, m_v, den_v,
               sem):
    c = lax.axis_index("c")
    s = lax.axis_index("s")
    wid = s * _NC + c
    E = eh_hbm.shape[0]
    ew = E // _NW
    base = wid * ew
    neg = jnp.full((16,), -1e30, jnp.float32)
    zer = jnp.zeros((16,), jnp.float32)

    def init(j, carry):
        sl = pl.ds(j * 16, 16)
        m_v[sl] = neg
        den_v[sl] = zer
        return carry

    lax.fori_loop(0, _NP // 16, init, 0)
    pltpu.sync_copy(dst_hbm.at[pl.ds(base, ew)], di_all)
    iota = lax.iota(jnp.int32, 16)

    def chunk(i, carry):
        b = base + i * _CK
        pltpu.sync_copy(src_hbm.at[pl.ds(b, _CK)], si_v)
        pltpu.sync_copy(dst_hbm.at[pl.ds(b, _CK)], di_v)
        pltpu.sync_copy(eh_hbm.at[pl.ds(b, _CK)], e_v)
        pltpu.async_copy(nh_hbm.at[si_v], s_v, sem).wait()
        pltpu.async_copy(nh_hbm.at[di_v], d_v, sem).wait()
        for g in range(_CK // 16):
            ridx = g * 16 + iota

            def cloop(cc, acc):
                cidx = (iota + cc) & 127
                sv = plsc.load_gather(s_v, [ridx, cidx])
                ev = plsc.load_gather(e_v, [ridx, cidx])
                dv = plsc.load_gather(d_v, [ridx, cidx])
                return acc + (sv + ev) * dv

            acc = lax.fori_loop(0, 128, cloop, jnp.zeros((16,), jnp.float32))
            attn_v[pl.ds(i * _CK + g * 16, 16)] = acc
        return carry

    lax.fori_loop(0, ew // _CK, chunk, 0)

    def mloop(j, carry):
        sl = pl.ds(j * 16, 16)
        dd16 = di_all[sl]
        av = attn_v[sl]

        def mcond(pending):
            return jnp.any(pending)

        def mbody(pending):
            cur = plsc.load_gather(m_v, [dd16])
            plsc.store_scatter(m_v, [dd16], jnp.maximum(cur, av),
                               mask=pending)
            cur2 = plsc.load_gather(m_v, [dd16])
            return pending & (cur2 < av)

        lax.while_loop(mcond, mbody, iota < 16)
        return carry

    lax.fori_loop(0, ew // 16, mloop, 0)

    def eloop(j, carry):
        sl = pl.ds(j * 16, 16)
        dd16 = di_all[sl]
        mv = plsc.load_gather(m_v, [dd16])
        ex = jnp.exp(attn_v[sl] - mv)
        plsc.addupdate_scatter(den_v, [dd16], ex)
        return carry

    lax.fori_loop(0, ew // 16, eloop, 0)
    pltpu.sync_copy(attn_v, attn_hbm.at[pl.ds(base, ew)])
    pltpu.sync_copy(m_v, maxp_hbm.at[pl.ds(wid * _NP, _NP)])
    pltpu.sync_copy(den_v, denp_hbm.at[pl.ds(wid * _NP, _NP)])


def _attn_sc(nh, eh, src, dst):
    E = eh.shape[0]
    ew = E // _NW
    mesh = plsc.VectorSubcoreMesh(core_axis_name="c", subcore_axis_name="s")
    f = pl.kernel(
        _attn_body,
        out_type=[
            jax.ShapeDtypeStruct((E,), jnp.float32),
            jax.ShapeDtypeStruct((_NW * _NP,), jnp.float32),
            jax.ShapeDtypeStruct((_NW * _NP,), jnp.float32),
        ],
        mesh=mesh,
        scratch_types=[
            pltpu.VMEM((_CK,), jnp.int32),
            pltpu.VMEM((_CK,), jnp.int32),
            pltpu.VMEM((_CK, 128), jnp.float32),
            pltpu.VMEM((_CK, 128), jnp.float32),
            pltpu.VMEM((_CK, 128), jnp.float32),
            pltpu.VMEM((ew,), jnp.float32),
            pltpu.VMEM((ew,), jnp.int32),
            pltpu.VMEM((_NP,), jnp.float32),
            pltpu.VMEM((_NP,), jnp.float32),
            pltpu.SemaphoreType.DMA,
        ],
        compiler_params=pltpu.CompilerParams(needs_layout_passes=False),
    )
    return f(nh, eh, src, dst)


_NHALF = 5056
_NPH = 5120


def _scatter_body(nh_hbm, attn_hbm, src_hbm, dst_hbm, maxp_hbm, denp_hbm,
                  nzp_hbm, M_hbm, D_hbm,
                  si_0, si_1, di_0, di_1, di2_v, at_0, at_1, a_v, s_0, s_1,
                  zbuf, mw_buf, dw_buf, M_v, D_v, nz_sh,
                  sem, semA0, semA1, semB0, semB1):
    si_l, di_l, at_l, s_l = [si_0, si_1], [di_0, di_1], [at_0, at_1], [s_0, s_1]
    semA = [semA0, semA1]
    semB = [semB0, semB1]
    c = lax.axis_index("c")
    s = lax.axis_index("s")
    wid = s * _NC + c
    E = attn_hbm.shape[0]
    ew = E // _NW
    base = wid * ew
    ns_base = s * _NPS
    iota = lax.iota(jnp.int32, 16)
    zer16 = jnp.zeros((16,), jnp.float32)
    nq = (_NPS + 15) // 16

    def minit(q, carry):
        lidx = q * 16 + iota
        msk = lidx < _NPS
        plsc.store_scatter(M_v, [ns_base + lidx],
                           jnp.full((16,), -1e30, jnp.float32), mask=msk)
        plsc.store_scatter(D_v, [ns_base + lidx], zer16, mask=msk)
        return carry

    lax.fori_loop(0, nq, minit, 0)

    def wmax(w, carry):
        pltpu.sync_copy(maxp_hbm.at[pl.ds(w * _NP + ns_base, _NPS)],
                        mw_buf.at[pl.ds(0, _NPS)])

        def q1(q, carry2):
            lidx = q * 16 + iota
            msk = lidx < _NPS
            idxs = ns_base + lidx
            cur = plsc.load_gather(M_v, [idxs], mask=msk)
            mw = plsc.load_gather(mw_buf, [lidx], mask=msk)
            plsc.store_scatter(M_v, [idxs], jnp.maximum(cur, mw), mask=msk)
            return carry2

        lax.fori_loop(0, nq, q1, 0)
        return carry

    lax.fori_loop(0, _NW, wmax, 0)

    def wden(w, carry):
        pltpu.sync_copy(maxp_hbm.at[pl.ds(w * _NP + ns_base, _NPS)],
                        mw_buf.at[pl.ds(0, _NPS)])
        pltpu.sync_copy(denp_hbm.at[pl.ds(w * _NP + ns_base, _NPS)],
                        dw_buf.at[pl.ds(0, _NPS)])

        def q2(q, carry2):
            lidx = q * 16 + iota
            msk = lidx < _NPS
            idxs = ns_base + lidx
            cur = plsc.load_gather(D_v, [idxs], mask=msk)
            mw = plsc.load_gather(mw_buf, [lidx], mask=msk)
            dw = plsc.load_gather(dw_buf, [lidx], mask=msk)
            mfin = plsc.load_gather(M_v, [idxs], mask=msk)
            plsc.store_scatter(D_v, [idxs], cur + dw * jnp.exp(mw - mfin),
                               mask=msk)
            return carry2

        lax.fori_loop(0, nq, q2, 0)
        return carry

    lax.fori_loop(0, _NW, wden, 0)
    pltpu.sync_copy(M_v.at[pl.ds(ns_base, _NPS)], M_hbm.at[pl.ds(ns_base, _NPS)])
    pltpu.sync_copy(D_v.at[pl.ds(ns_base, _NPS)], D_hbm.at[pl.ds(ns_base, _NPS)])
    plsc.subcore_barrier()
    pltpu.sync_copy(M_hbm, M_v.at[pl.ds(0, _NP)])
    pltpu.sync_copy(D_hbm, D_v.at[pl.ds(0, _NP)])

    def zero_zbuf():
        msk8 = iota < 8

        def zcol(cc, carry2):
            cidx = iota * 0 + cc
            plsc.store_scatter(zbuf, [iota, cidx], zer16, mask=msk8)
            return carry2

        lax.fori_loop(0, 128, zcol, 0)

    ws_base = s * (_NPH // _NS)
    for p in range(2):
        zero_zbuf()
        for q in range(_NPH // _NS // 8):
            pltpu.sync_copy(zbuf, nz_sh.at[pl.ds(ws_base + q * 8, 8)])
        plsc.subcore_barrier()

        NCH = ew // _CK

        def issue_idx(ci, t):
            b2 = base + ci * _CK
            pltpu.async_copy(src_hbm.at[pl.ds(b2, _CK)], si_l[t], semA[t])
            pltpu.async_copy(dst_hbm.at[pl.ds(b2, _CK)], di_l[t], semA[t])
            pltpu.async_copy(attn_hbm.at[pl.ds(b2, _CK)], at_l[t], semA[t])

        def wait_idx(ci, t):
            b2 = base + ci * _CK
            pltpu.make_async_copy(src_hbm.at[pl.ds(b2, _CK)], si_l[t],
                                  semA[t]).wait()
            pltpu.make_async_copy(dst_hbm.at[pl.ds(b2, _CK)], di_l[t],
                                  semA[t]).wait()
            pltpu.make_async_copy(attn_hbm.at[pl.ds(b2, _CK)], at_l[t],
                                  semA[t]).wait()

        def compute_scale(t):
            for g in range(_CK // 16):
                sl = pl.ds(g * 16, 16)
                ridx = g * 16 + iota
                dl = di_l[t][sl]
                mv = plsc.load_gather(M_v, [dl])
                dv = plsc.load_gather(D_v, [dl])
                a16 = jnp.exp(at_l[t][sl] - mv) / dv
                loc = dl - p * _NHALF
                ok = (loc >= 0) & (loc < _NHALF)
                di2_v[sl] = jnp.where(ok, loc, _NHALF)

                def ccol(cc, carry2):
                    cidx = (iota + cc) & 127
                    sv = plsc.load_gather(s_l[t], [ridx, cidx])
                    plsc.store_scatter(s_l[t], [ridx, cidx], sv * a16)
                    return carry2

                lax.fori_loop(0, 128, ccol, 0)

        def one_iter(ci, t):
            u = 1 - t
            pltpu.make_async_copy(nh_hbm.at[si_l[t]], s_l[t],
                                  semB[t]).wait()
            compute_scale(t)
            wait_idx(ci + 1, u)
            pltpu.async_copy(nh_hbm.at[si_l[u]], s_l[u], semB[u])
            issue_idx(jnp.minimum(ci + 2, NCH - 1), t)
            pltpu.sync_copy(s_l[t], nz_sh.at[di2_v], add=True)

        issue_idx(0, 0)
        wait_idx(0, 0)
        pltpu.async_copy(nh_hbm.at[si_l[0]], s_l[0], semB[0])
        issue_idx(1, 1)

        def pair(k, carry):
            one_iter(2 * k, 0)
            one_iter(2 * k + 1, 1)
            return carry

        lax.fori_loop(0, (NCH - 1) // 2, pair, 0)
        pltpu.make_async_copy(nh_hbm.at[si_l[0]], s_l[0], semB[0]).wait()
        compute_scale(0)
        pltpu.sync_copy(s_l[0], nz_sh.at[di2_v], add=True)
        wait_idx(NCH - 1, 1)
        plsc.subcore_barrier()
        for q in range(_NPH // _NS // 8):
            pltpu.sync_copy(nz_sh.at[pl.ds(ws_base + q * 8, 8)], zbuf)
            pltpu.sync_copy(zbuf,
                            nzp_hbm.at[c, p, pl.ds(ws_base + q * 8, 8)])


def _scatter_sc(nh, attn, src, dst, maxp, denp):
    E = attn.shape[0]
    mesh = plsc.VectorSubcoreMesh(core_axis_name="c", subcore_axis_name="s")
    f = pl.kernel(
        _scatter_body,
        out_type=[
            jax.ShapeDtypeStruct((_NC, 2, _NPH, 128), jnp.float32),
            jax.ShapeDtypeStruct((_NP,), jnp.float32),
            jax.ShapeDtypeStruct((_NP,), jnp.float32),
        ],
        mesh=mesh,
        scratch_types=[
            pltpu.VMEM((_CK,), jnp.int32),
            pltpu.VMEM((_CK,), jnp.int32),
            pltpu.VMEM((_CK,), jnp.int32),
            pltpu.VMEM((_CK,), jnp.int32),
            pltpu.VMEM((_CK,), jnp.int32),
            pltpu.VMEM((_CK,), jnp.float32),
            pltpu.VMEM((_CK,), jnp.float32),
            pltpu.VMEM((_CK,), jnp.float32),
            pltpu.VMEM((_CK, 128), jnp.float32),
            pltpu.VMEM((_CK, 128), jnp.float32),
            pltpu.VMEM((8, 128), jnp.float32),
            pltpu.VMEM((640,), jnp.float32),
            pltpu.VMEM((640,), jnp.float32),
            pltpu.VMEM((_NP + 32,), jnp.float32),
            pltpu.VMEM((_NP + 32,), jnp.float32),
            pltpu.VMEM_SHARED((_NPH, 128), jnp.float32),
            pltpu.SemaphoreType.DMA,
            pltpu.SemaphoreType.DMA,
            pltpu.SemaphoreType.DMA,
            pltpu.SemaphoreType.DMA,
            pltpu.SemaphoreType.DMA,
        ],
        compiler_params=pltpu.CompilerParams(needs_layout_passes=False),
    )
    return f(nh, attn, src, dst, maxp, denp)[0]


def _epre_body(epsp_hbm, eh_hbm, nz_hbm, src_hbm, dst_hbm, out_hbm,
               epsp_v, eh_v, s_v, d_v, si_v, di_v, sem):
    c = lax.axis_index("c")
    s = lax.axis_index("s")
    wid = s * _NC + c
    E = eh_hbm.shape[0]
    ew = E // _NW
    base = wid * ew
    pltpu.sync_copy(epsp_hbm, epsp_v)
    eps_sl = [epsp_v[pl.ds(t * 16, 16)] for t in range(8)]

    def chunk(i, carry):
        b = base + i * _CK
        pltpu.sync_copy(src_hbm.at[pl.ds(b, _CK)], si_v)
        pltpu.sync_copy(dst_hbm.at[pl.ds(b, _CK)], di_v)
        pltpu.sync_copy(eh_hbm.at[pl.ds(b, _CK)], eh_v)
        pltpu.async_copy(nz_hbm.at[si_v], s_v, sem).wait()
        pltpu.async_copy(nz_hbm.at[di_v], d_v, sem).wait()

        def row(j, carry2):
            for t in range(8):
                sl = pl.ds(t * 16, 16)
                eh_v[j, sl] = (eh_v[j, sl] * eps_sl[t]
                               + s_v[j, sl] - d_v[j, sl])
            return carry2

        lax.fori_loop(0, _CK, row, 0)
        pltpu.sync_copy(eh_v, out_hbm.at[pl.ds(b, _CK)])
        return carry

    lax.fori_loop(0, ew // _CK, chunk, 0)


def _epre_sc(epsp, eh, nz, src, dst):
    E = eh.shape[0]
    mesh = plsc.VectorSubcoreMesh(core_axis_name="c", subcore_axis_name="s")
    f = pl.kernel(
        _epre_body,
        out_type=jax.ShapeDtypeStruct((E, 128), jnp.float32),
        mesh=mesh,
        scratch_types=[
            pltpu.VMEM((128,), jnp.float32),
            pltpu.VMEM((_CK, 128), jnp.float32),
            pltpu.VMEM((_CK, 128), jnp.float32),
            pltpu.VMEM((_CK, 128), jnp.float32),
            pltpu.VMEM((_CK,), jnp.int32),
            pltpu.VMEM((_CK,), jnp.int32),
            pltpu.SemaphoreType.DMA,
        ],
    )
    return f(epsp, eh, nz, src, dst)


def _mlp_stats_body(x_ref, w1_ref, b1_ref, w2_ref, b2_ref,
                    y_ref, s1_ref, s2_ref):
    i = pl.program_id(0)
    x = x_ref[...]
    h = jnp.maximum(
        jnp.dot(x, w1_ref[...], preferred_element_type=jnp.float32)
        + b1_ref[...], 0.0)
    y = (jnp.dot(h, w2_ref[...], preferred_element_type=jnp.float32)
         + b2_ref[...])
    y_ref[...] = y

    @pl.when(i == 0)
    def _init():
        s1_ref[...] = jnp.zeros_like(s1_ref)
        s2_ref[...] = jnp.zeros_like(s2_ref)

    s1_ref[...] += jnp.sum(y, axis=0, keepdims=True)
    s2_ref[...] += jnp.sum(y * y, axis=0, keepdims=True)


def _mlp_stats(x, w1, b1, w2, b2, block):
    n = x.shape[0]
    grid = n // block
    y, s1, s2 = pl.pallas_call(
        _mlp_stats_body,
        grid=(grid,),
        in_specs=[
            pl.BlockSpec((block, 128), lambda i: (i, 0)),
            pl.BlockSpec((128, 128), lambda i: (0, 0)),
            pl.BlockSpec((1, 128), lambda i: (0, 0)),
            pl.BlockSpec((128, 128), lambda i: (0, 0)),
            pl.BlockSpec((1, 128), lambda i: (0, 0)),
        ],
        out_specs=[
            pl.BlockSpec((block, 128), lambda i: (i, 0)),
            pl.BlockSpec((1, 128), lambda i: (0, 0)),
            pl.BlockSpec((1, 128), lambda i: (0, 0)),
        ],
        out_shape=[
            jax.ShapeDtypeStruct((n, 128), jnp.float32),
            jax.ShapeDtypeStruct((1, 128), jnp.float32),
            jax.ShapeDtypeStruct((1, 128), jnp.float32),
        ],
    )(x, w1, b1.reshape(1, 128), w2, b2.reshape(1, 128))
    return y, s1, s2


def _bn_body(y_ref, mu_ref, rstd_ref, gamma_ref, beta_ref, o_ref):
    o_ref[...] = ((y_ref[...] - mu_ref[...]) * rstd_ref[...]
                  * gamma_ref[...] + beta_ref[...])


def _bn_apply(y, mu, rstd, gamma, beta, block):
    n = y.shape[0]
    return pl.pallas_call(
        _bn_body,
        grid=(n // block,),
        in_specs=[
            pl.BlockSpec((block, 128), lambda i: (i, 0)),
            pl.BlockSpec((1, 128), lambda i: (0, 0)),
            pl.BlockSpec((1, 128), lambda i: (0, 0)),
            pl.BlockSpec((1, 128), lambda i: (0, 0)),
            pl.BlockSpec((1, 128), lambda i: (0, 0)),
        ],
        out_specs=pl.BlockSpec((block, 128), lambda i: (i, 0)),
        out_shape=jax.ShapeDtypeStruct((n, 128), jnp.float32),
    )(y, mu, rstd, gamma.reshape(1, 128), beta.reshape(1, 128))


def _mlp_bn(x, w1, b1, w2, b2, gamma, beta, block):
    n = x.shape[0]
    y, s1, s2 = _mlp_stats(x, w1, b1, w2, b2, block)
    mu = s1 / n
    var = s2 / n - mu * mu
    rstd = jax.lax.rsqrt(var + 1e-5)
    return _bn_apply(y, mu, rstd, gamma, beta, block)


def kernel(nh, eh, edge_index,
           nf_W1, nf_b1, nf_W2, nf_b2, nf_eps, nf_gamma, nf_beta,
           ef_W1, ef_b1, ef_W2, ef_b2, ef_eps, ef_gamma, ef_beta):
    N = nh.shape[0]
    src, dst = edge_index[0], edge_index[1]
    attn, maxp, denp = _attn_sc(nh, eh, src, dst)
    nzp = _scatter_sc(nh, attn, src, dst, maxp, denp)
    nz = jnp.concatenate([(nzp[0, 0] + nzp[1, 0])[:_NHALF],
                          (nzp[0, 1] + nzp[1, 1])[:N - _NHALF]], axis=0)
    n_pre = (1.0 + nf_eps) * nh + nz
    e_pre = _epre_sc(1.0 + ef_eps, eh, nz, src, dst)
    n_h = _mlp_bn(n_pre, nf_W1, nf_b1, nf_W2, nf_b2, nf_gamma, nf_beta, 1000)
    e_h = _mlp_bn(e_pre, ef_W1, ef_b1, ef_W2, ef_b2, ef_gamma, ef_beta, 1000)
    return (n_h, e_h)


# speedup vs baseline: 2.4679x; 1.1188x over previous
import functools

import jax
import jax.numpy as jnp
from jax import lax
from jax.experimental import pallas as pl
from jax.experimental.pallas import tpu as pltpu
from jax.experimental.pallas import tpu_sc as plsc

_NC = 2
_NS = 16
_NW = _NC * _NS
_CK = 80


_NP = 10112
_NPS = _NP // _NS


def _attn_body(nh_hbm, eh_hbm, src_hbm, dst_hbm,
               attn_hbm, maxp_hbm, denp_hbm,
               si_0, si_1, di_0, di_1, s_0, s_1, d_0, d_1, e_0, e_1,
               attn_v, di_all, m_v, den_v,
               semA0, semA1, semB0, semB1):
    si_l, di_l = [si_0, si_1], [di_0, di_1]
    s_l, d_l, e_l = [s_0, s_1], [d_0, d_1], [e_0, e_1]
    semA = [semA0, semA1]
    semB = [semB0, semB1]
    c = lax.axis_index("c")
    s = lax.axis_index("s")
    wid = s * _NC + c
    E = eh_hbm.shape[0]
    ew = E // _NW
    base = wid * ew
    neg = jnp.full((16,), -1e30, jnp.float32)
    zer = jnp.zeros((16,), jnp.float32)

    def init(j, carry):
        sl = pl.ds(j * 16, 16)
        m_v[sl] = neg
        den_v[sl] = zer
        return carry

    lax.fori_loop(0, _NP // 16, init, 0)
    pltpu.sync_copy(dst_hbm.at[pl.ds(base, ew)], di_all)
    iota = lax.iota(jnp.int32, 16)

    NCH = ew // _CK

    def issue_idx(ci, t):
        b2 = base + ci * _CK
        pltpu.async_copy(src_hbm.at[pl.ds(b2, _CK)], si_l[t], semA[t])
        pltpu.async_copy(dst_hbm.at[pl.ds(b2, _CK)], di_l[t], semA[t])
        pltpu.async_copy(eh_hbm.at[pl.ds(b2, _CK)], e_l[t], semA[t])

    def wait_idx(ci, t):
        b2 = base + ci * _CK
        pltpu.make_async_copy(src_hbm.at[pl.ds(b2, _CK)], si_l[t],
                              semA[t]).wait()
        pltpu.make_async_copy(dst_hbm.at[pl.ds(b2, _CK)], di_l[t],
                              semA[t]).wait()
        pltpu.make_async_copy(eh_hbm.at[pl.ds(b2, _CK)], e_l[t],
                              semA[t]).wait()

    def issue_gather(t):
        pltpu.async_copy(nh_hbm.at[si_l[t]], s_l[t], semB[t])
        pltpu.async_copy(nh_hbm.at[di_l[t]], d_l[t], semB[t])

    def wait_gather(t):
        pltpu.make_async_copy(nh_hbm.at[si_l[t]], s_l[t], semB[t]).wait()
        pltpu.make_async_copy(nh_hbm.at[di_l[t]], d_l[t], semB[t]).wait()

    def compute(ci, t):
        for g in range(_CK // 16):
            ridx = g * 16 + iota

            def cloop(cc, acc):
                cidx = (iota + cc) & 127
                sv = plsc.load_gather(s_l[t], [ridx, cidx])
                ev = plsc.load_gather(e_l[t], [ridx, cidx])
                dv = plsc.load_gather(d_l[t], [ridx, cidx])
                return acc + (sv + ev) * dv

            acc = lax.fori_loop(0, 128, cloop, jnp.zeros((16,), jnp.float32))
            attn_v[pl.ds(ci * _CK + g * 16, 16)] = acc

    def one_iter(ci, t):
        u = 1 - t
        wait_gather(t)
        wait_idx(ci + 1, u)
        issue_gather(u)
        compute(ci, t)
        issue_idx(jnp.minimum(ci + 2, NCH - 1), t)

    issue_idx(0, 0)
    wait_idx(0, 0)
    issue_gather(0)
    issue_idx(1, 1)

    def pair(k, carry):
        one_iter(2 * k, 0)
        one_iter(2 * k + 1, 1)
        return carry

    lax.fori_loop(0, (NCH - 1) // 2, pair, 0)
    wait_gather(0)
    compute(NCH - 1, 0)
    wait_idx(NCH - 1, 1)

    def mloop(j, carry):
        sl = pl.ds(j * 16, 16)
        dd16 = di_all[sl]
        av = attn_v[sl]

        def mcond(pending):
            return jnp.any(pending)

        def mbody(pending):
            cur = plsc.load_gather(m_v, [dd16])
            plsc.store_scatter(m_v, [dd16], jnp.maximum(cur, av),
                               mask=pending)
            cur2 = plsc.load_gather(m_v, [dd16])
            return pending & (cur2 < av)

        lax.while_loop(mcond, mbody, iota < 16)
        return carry

    lax.fori_loop(0, ew // 16, mloop, 0)

    def eloop(j, carry):
        sl = pl.ds(j * 16, 16)
        dd16 = di_all[sl]
        mv = plsc.load_gather(m_v, [dd16])
        ex = jnp.exp(attn_v[sl] - mv)
        plsc.addupdate_scatter(den_v, [dd16], ex)
        return carry

    lax.fori_loop(0, ew // 16, eloop, 0)
    pltpu.sync_copy(attn_v, attn_hbm.at[pl.ds(base, ew)])
    pltpu.sync_copy(m_v, maxp_hbm.at[pl.ds(wid * _NP, _NP)])
    pltpu.sync_copy(den_v, denp_hbm.at[pl.ds(wid * _NP, _NP)])


def _attn_sc(nh, eh, src, dst):
    E = eh.shape[0]
    ew = E // _NW
    mesh = plsc.VectorSubcoreMesh(core_axis_name="c", subcore_axis_name="s")
    f = pl.kernel(
        _attn_body,
        out_type=[
            jax.ShapeDtypeStruct((E,), jnp.float32),
            jax.ShapeDtypeStruct((_NW * _NP,), jnp.float32),
            jax.ShapeDtypeStruct((_NW * _NP,), jnp.float32),
        ],
        mesh=mesh,
        scratch_types=[
            pltpu.VMEM((_CK,), jnp.int32),
            pltpu.VMEM((_CK,), jnp.int32),
            pltpu.VMEM((_CK,), jnp.int32),
            pltpu.VMEM((_CK,), jnp.int32),
            pltpu.VMEM((_CK, 128), jnp.float32),
            pltpu.VMEM((_CK, 128), jnp.float32),
            pltpu.VMEM((_CK, 128), jnp.float32),
            pltpu.VMEM((_CK, 128), jnp.float32),
            pltpu.VMEM((_CK, 128), jnp.float32),
            pltpu.VMEM((_CK, 128), jnp.float32),
            pltpu.VMEM((ew,), jnp.float32),
            pltpu.VMEM((ew,), jnp.int32),
            pltpu.VMEM((_NP,), jnp.float32),
            pltpu.VMEM((_NP,), jnp.float32),
            pltpu.SemaphoreType.DMA,
            pltpu.SemaphoreType.DMA,
            pltpu.SemaphoreType.DMA,
            pltpu.SemaphoreType.DMA,
        ],
        compiler_params=pltpu.CompilerParams(needs_layout_passes=False),
    )
    return f(nh, eh, src, dst)


_NHALF = 5056
_NPH = 5120


def _scatter_body(nh_hbm, attn_hbm, src_hbm, dst_hbm, maxp_hbm, denp_hbm,
                  nzp_hbm, M_hbm, D_hbm,
                  si_0, si_1, di_0, di_1, di2_v, at_0, at_1, a_v, s_0, s_1,
                  zbuf, mw_buf, dw_buf, M_v, D_v, nz_sh,
                  sem, semA0, semA1, semB0, semB1):
    si_l, di_l, at_l, s_l = [si_0, si_1], [di_0, di_1], [at_0, at_1], [s_0, s_1]
    semA = [semA0, semA1]
    semB = [semB0, semB1]
    c = lax.axis_index("c")
    s = lax.axis_index("s")
    wid = s * _NC + c
    E = attn_hbm.shape[0]
    ew = E // _NW
    base = wid * ew
    ns_base = s * _NPS
    iota = lax.iota(jnp.int32, 16)
    zer16 = jnp.zeros((16,), jnp.float32)
    nq = (_NPS + 15) // 16

    def minit(q, carry):
        lidx = q * 16 + iota
        msk = lidx < _NPS
        plsc.store_scatter(M_v, [ns_base + lidx],
                           jnp.full((16,), -1e30, jnp.float32), mask=msk)
        plsc.store_scatter(D_v, [ns_base + lidx], zer16, mask=msk)
        return carry

    lax.fori_loop(0, nq, minit, 0)

    def wmax(w, carry):
        pltpu.sync_copy(maxp_hbm.at[pl.ds(w * _NP + ns_base, _NPS)],
                        mw_buf.at[pl.ds(0, _NPS)])

        def q1(q, carry2):
            lidx = q * 16 + iota
            msk = lidx < _NPS
            idxs = ns_base + lidx
            cur = plsc.load_gather(M_v, [idxs], mask=msk)
            mw = plsc.load_gather(mw_buf, [lidx], mask=msk)
            plsc.store_scatter(M_v, [idxs], jnp.maximum(cur, mw), mask=msk)
            return carry2

        lax.fori_loop(0, nq, q1, 0)
        return carry

    lax.fori_loop(0, _NW, wmax, 0)

    def wden(w, carry):
        pltpu.sync_copy(maxp_hbm.at[pl.ds(w * _NP + ns_base, _NPS)],
                        mw_buf.at[pl.ds(0, _NPS)])
        pltpu.sync_copy(denp_hbm.at[pl.ds(w * _NP + ns_base, _NPS)],
                        dw_buf.at[pl.ds(0, _NPS)])

        def q2(q, carry2):
            lidx = q * 16 + iota
            msk = lidx < _NPS
            idxs = ns_base + lidx
            cur = plsc.load_gather(D_v, [idxs], mask=msk)
            mw = plsc.load_gather(mw_buf, [lidx], mask=msk)
            dw = plsc.load_gather(dw_buf, [lidx], mask=msk)
            mfin = plsc.load_gather(M_v, [idxs], mask=msk)
            plsc.store_scatter(D_v, [idxs], cur + dw * jnp.exp(mw - mfin),
                               mask=msk)
            return carry2

        lax.fori_loop(0, nq, q2, 0)
        return carry

    lax.fori_loop(0, _NW, wden, 0)
    pltpu.sync_copy(M_v.at[pl.ds(ns_base, _NPS)], M_hbm.at[pl.ds(ns_base, _NPS)])
    pltpu.sync_copy(D_v.at[pl.ds(ns_base, _NPS)], D_hbm.at[pl.ds(ns_base, _NPS)])
    plsc.subcore_barrier()
    pltpu.sync_copy(M_hbm, M_v.at[pl.ds(0, _NP)])
    pltpu.sync_copy(D_hbm, D_v.at[pl.ds(0, _NP)])

    def zero_zbuf():
        msk8 = iota < 8

        def zcol(cc, carry2):
            cidx = iota * 0 + cc
            plsc.store_scatter(zbuf, [iota, cidx], zer16, mask=msk8)
            return carry2

        lax.fori_loop(0, 128, zcol, 0)

    ws_base = s * (_NPH // _NS)
    for p in range(2):
        zero_zbuf()
        for q in range(_NPH // _NS // 8):
            pltpu.sync_copy(zbuf, nz_sh.at[pl.ds(ws_base + q * 8, 8)])
        plsc.subcore_barrier()

        NCH = ew // _CK

        def issue_idx(ci, t):
            b2 = base + ci * _CK
            pltpu.async_copy(src_hbm.at[pl.ds(b2, _CK)], si_l[t], semA[t])
            pltpu.async_copy(dst_hbm.at[pl.ds(b2, _CK)], di_l[t], semA[t])
            pltpu.async_copy(attn_hbm.at[pl.ds(b2, _CK)], at_l[t], semA[t])

        def wait_idx(ci, t):
            b2 = base + ci * _CK
            pltpu.make_async_copy(src_hbm.at[pl.ds(b2, _CK)], si_l[t],
                                  semA[t]).wait()
            pltpu.make_async_copy(dst_hbm.at[pl.ds(b2, _CK)], di_l[t],
                                  semA[t]).wait()
            pltpu.make_async_copy(attn_hbm.at[pl.ds(b2, _CK)], at_l[t],
                                  semA[t]).wait()

        def compute_scale(t):
            for g in range(_CK // 16):
                sl = pl.ds(g * 16, 16)
                ridx = g * 16 + iota
                dl = di_l[t][sl]
                mv = plsc.load_gather(M_v, [dl])
                dv = plsc.load_gather(D_v, [dl])
                a16 = jnp.exp(at_l[t][sl] - mv) / dv
                loc = dl - p * _NHALF
                ok = (loc >= 0) & (loc < _NHALF)
                di2_v[sl] = jnp.where(ok, loc, _NHALF)

                def ccol(cc, carry2):
                    cidx = (iota + cc) & 127
                    sv = plsc.load_gather(s_l[t], [ridx, cidx])
                    plsc.store_scatter(s_l[t], [ridx, cidx], sv * a16)
                    return carry2

                lax.fori_loop(0, 128, ccol, 0)

        def one_iter(ci, t):
            u = 1 - t
            pltpu.make_async_copy(nh_hbm.at[si_l[t]], s_l[t],
                                  semB[t]).wait()
            compute_scale(t)
            wait_idx(ci + 1, u)
            pltpu.async_copy(nh_hbm.at[si_l[u]], s_l[u], semB[u])
            issue_idx(jnp.minimum(ci + 2, NCH - 1), t)
            pltpu.sync_copy(s_l[t], nz_sh.at[di2_v], add=True)

        issue_idx(0, 0)
        wait_idx(0, 0)
        pltpu.async_copy(nh_hbm.at[si_l[0]], s_l[0], semB[0])
        issue_idx(1, 1)

        def pair(k, carry):
            one_iter(2 * k, 0)
            one_iter(2 * k + 1, 1)
            return carry

        lax.fori_loop(0, (NCH - 1) // 2, pair, 0)
        pltpu.make_async_copy(nh_hbm.at[si_l[0]], s_l[0], semB[0]).wait()
        compute_scale(0)
        pltpu.sync_copy(s_l[0], nz_sh.at[di2_v], add=True)
        wait_idx(NCH - 1, 1)
        plsc.subcore_barrier()
        for q in range(_NPH // _NS // 8):
            pltpu.sync_copy(nz_sh.at[pl.ds(ws_base + q * 8, 8)], zbuf)
            pltpu.sync_copy(zbuf,
                            nzp_hbm.at[c, p, pl.ds(ws_base + q * 8, 8)])


def _scatter_sc(nh, attn, src, dst, maxp, denp):
    E = attn.shape[0]
    mesh = plsc.VectorSubcoreMesh(core_axis_name="c", subcore_axis_name="s")
    f = pl.kernel(
        _scatter_body,
        out_type=[
            jax.ShapeDtypeStruct((_NC, 2, _NPH, 128), jnp.float32),
            jax.ShapeDtypeStruct((_NP,), jnp.float32),
            jax.ShapeDtypeStruct((_NP,), jnp.float32),
        ],
        mesh=mesh,
        scratch_types=[
            pltpu.VMEM((_CK,), jnp.int32),
            pltpu.VMEM((_CK,), jnp.int32),
            pltpu.VMEM((_CK,), jnp.int32),
            pltpu.VMEM((_CK,), jnp.int32),
            pltpu.VMEM((_CK,), jnp.int32),
            pltpu.VMEM((_CK,), jnp.float32),
            pltpu.VMEM((_CK,), jnp.float32),
            pltpu.VMEM((_CK,), jnp.float32),
            pltpu.VMEM((_CK, 128), jnp.float32),
            pltpu.VMEM((_CK, 128), jnp.float32),
            pltpu.VMEM((8, 128), jnp.float32),
            pltpu.VMEM((640,), jnp.float32),
            pltpu.VMEM((640,), jnp.float32),
            pltpu.VMEM((_NP + 32,), jnp.float32),
            pltpu.VMEM((_NP + 32,), jnp.float32),
            pltpu.VMEM_SHARED((_NPH, 128), jnp.float32),
            pltpu.SemaphoreType.DMA,
            pltpu.SemaphoreType.DMA,
            pltpu.SemaphoreType.DMA,
            pltpu.SemaphoreType.DMA,
            pltpu.SemaphoreType.DMA,
        ],
        compiler_params=pltpu.CompilerParams(needs_layout_passes=False),
    )
    return f(nh, attn, src, dst, maxp, denp)[0]


def _epre_body(epsp_hbm, eh_hbm, nz_hbm, src_hbm, dst_hbm, out_hbm,
               epsp_v, eh_v, s_v, d_v, si_v, di_v, sem):
    c = lax.axis_index("c")
    s = lax.axis_index("s")
    wid = s * _NC + c
    E = eh_hbm.shape[0]
    ew = E // _NW
    base = wid * ew
    pltpu.sync_copy(epsp_hbm, epsp_v)
    eps_sl = [epsp_v[pl.ds(t * 16, 16)] for t in range(8)]

    def chunk(i, carry):
        b = base + i * _CK
        pltpu.sync_copy(src_hbm.at[pl.ds(b, _CK)], si_v)
        pltpu.sync_copy(dst_hbm.at[pl.ds(b, _CK)], di_v)
        pltpu.sync_copy(eh_hbm.at[pl.ds(b, _CK)], eh_v)
        pltpu.async_copy(nz_hbm.at[si_v], s_v, sem).wait()
        pltpu.async_copy(nz_hbm.at[di_v], d_v, sem).wait()

        def row(j, carry2):
            for t in range(8):
                sl = pl.ds(t * 16, 16)
                eh_v[j, sl] = (eh_v[j, sl] * eps_sl[t]
                               + s_v[j, sl] - d_v[j, sl])
            return carry2

        lax.fori_loop(0, _CK, row, 0)
        pltpu.sync_copy(eh_v, out_hbm.at[pl.ds(b, _CK)])
        return carry

    lax.fori_loop(0, ew // _CK, chunk, 0)


def _epre_sc(epsp, eh, nz, src, dst):
    E = eh.shape[0]
    mesh = plsc.VectorSubcoreMesh(core_axis_name="c", subcore_axis_name="s")
    f = pl.kernel(
        _epre_body,
        out_type=jax.ShapeDtypeStruct((E, 128), jnp.float32),
        mesh=mesh,
        scratch_types=[
            pltpu.VMEM((128,), jnp.float32),
            pltpu.VMEM((_CK, 128), jnp.float32),
            pltpu.VMEM((_CK, 128), jnp.float32),
            pltpu.VMEM((_CK, 128), jnp.float32),
            pltpu.VMEM((_CK,), jnp.int32),
            pltpu.VMEM((_CK,), jnp.int32),
            pltpu.SemaphoreType.DMA,
        ],
    )
    return f(epsp, eh, nz, src, dst)


def _mlp_stats_body(x_ref, w1_ref, b1_ref, w2_ref, b2_ref,
                    y_ref, s1_ref, s2_ref):
    i = pl.program_id(0)
    x = x_ref[...]
    h = jnp.maximum(
        jnp.dot(x, w1_ref[...], preferred_element_type=jnp.float32)
        + b1_ref[...], 0.0)
    y = (jnp.dot(h, w2_ref[...], preferred_element_type=jnp.float32)
         + b2_ref[...])
    y_ref[...] = y

    @pl.when(i == 0)
    def _init():
        s1_ref[...] = jnp.zeros_like(s1_ref)
        s2_ref[...] = jnp.zeros_like(s2_ref)

    s1_ref[...] += jnp.sum(y, axis=0, keepdims=True)
    s2_ref[...] += jnp.sum(y * y, axis=0, keepdims=True)


def _mlp_stats(x, w1, b1, w2, b2, block):
    n = x.shape[0]
    grid = n // block
    y, s1, s2 = pl.pallas_call(
        _mlp_stats_body,
        grid=(grid,),
        in_specs=[
            pl.BlockSpec((block, 128), lambda i: (i, 0)),
            pl.BlockSpec((128, 128), lambda i: (0, 0)),
            pl.BlockSpec((1, 128), lambda i: (0, 0)),
            pl.BlockSpec((128, 128), lambda i: (0, 0)),
            pl.BlockSpec((1, 128), lambda i: (0, 0)),
        ],
        out_specs=[
            pl.BlockSpec((block, 128), lambda i: (i, 0)),
            pl.BlockSpec((1, 128), lambda i: (0, 0)),
            pl.BlockSpec((1, 128), lambda i: (0, 0)),
        ],
        out_shape=[
            jax.ShapeDtypeStruct((n, 128), jnp.float32),
            jax.ShapeDtypeStruct((1, 128), jnp.float32),
            jax.ShapeDtypeStruct((1, 128), jnp.float32),
        ],
    )(x, w1, b1.reshape(1, 128), w2, b2.reshape(1, 128))
    return y, s1, s2


def _bn_body(y_ref, mu_ref, rstd_ref, gamma_ref, beta_ref, o_ref):
    o_ref[...] = ((y_ref[...] - mu_ref[...]) * rstd_ref[...]
                  * gamma_ref[...] + beta_ref[...])


def _bn_apply(y, mu, rstd, gamma, beta, block):
    n = y.shape[0]
    return pl.pallas_call(
        _bn_body,
        grid=(n // block,),
        in_specs=[
            pl.BlockSpec((block, 128), lambda i: (i, 0)),
            pl.BlockSpec((1, 128), lambda i: (0, 0)),
            pl.BlockSpec((1, 128), lambda i: (0, 0)),
            pl.BlockSpec((1, 128), lambda i: (0, 0)),
            pl.BlockSpec((1, 128), lambda i: (0, 0)),
        ],
        out_specs=pl.BlockSpec((block, 128), lambda i: (i, 0)),
        out_shape=jax.ShapeDtypeStruct((n, 128), jnp.float32),
    )(y, mu, rstd, gamma.reshape(1, 128), beta.reshape(1, 128))


def _mlp_bn(x, w1, b1, w2, b2, gamma, beta, block):
    n = x.shape[0]
    y, s1, s2 = _mlp_stats(x, w1, b1, w2, b2, block)
    mu = s1 / n
    var = s2 / n - mu * mu
    rstd = jax.lax.rsqrt(var + 1e-5)
    return _bn_apply(y, mu, rstd, gamma, beta, block)


def kernel(nh, eh, edge_index,
           nf_W1, nf_b1, nf_W2, nf_b2, nf_eps, nf_gamma, nf_beta,
           ef_W1, ef_b1, ef_W2, ef_b2, ef_eps, ef_gamma, ef_beta):
    N = nh.shape[0]
    src, dst = edge_index[0], edge_index[1]
    attn, maxp, denp = _attn_sc(nh, eh, src, dst)
    nzp = _scatter_sc(nh, attn, src, dst, maxp, denp)
    nz = jnp.concatenate([(nzp[0, 0] + nzp[1, 0])[:_NHALF],
                          (nzp[0, 1] + nzp[1, 1])[:N - _NHALF]], axis=0)
    n_pre = (1.0 + nf_eps) * nh + nz
    e_pre = _epre_sc(1.0 + ef_eps, eh, nz, src, dst)
    n_h = _mlp_bn(n_pre, nf_W1, nf_b1, nf_W2, nf_b2, nf_gamma, nf_beta, 1000)
    e_h = _mlp_bn(e_pre, ef_W1, ef_b1, ef_W2, ef_b2, ef_gamma, ef_beta, 1000)
    return (n_h, e_h)


# speedup vs baseline: 2.8050x; 1.1366x over previous
import functools

import jax
import jax.numpy as jnp
from jax import lax
from jax.experimental import pallas as pl
from jax.experimental.pallas import tpu as pltpu
from jax.experimental.pallas import tpu_sc as plsc

_NC = 2
_NS = 16
_NW = _NC * _NS
_CK = 80


_NP = 10112
_NPS = _NP // _NS


def _attn_body(nh_hbm, eh_hbm, src_hbm, dst_hbm,
               attn_hbm, maxp_hbm, denp_hbm,
               si_0, si_1, di_0, di_1, s_0, s_1, d_0, d_1, e_0, e_1,
               attn_v, di_all, m_v, den_v,
               semA0, semA1, semB0, semB1):
    si_l, di_l = [si_0, si_1], [di_0, di_1]
    s_l, d_l, e_l = [s_0, s_1], [d_0, d_1], [e_0, e_1]
    semA = [semA0, semA1]
    semB = [semB0, semB1]
    c = lax.axis_index("c")
    s = lax.axis_index("s")
    wid = s * _NC + c
    E = eh_hbm.shape[0]
    ew = E // _NW
    base = wid * ew
    neg = jnp.full((16,), -1e30, jnp.float32)
    zer = jnp.zeros((16,), jnp.float32)

    def init(j, carry):
        sl = pl.ds(j * 16, 16)
        m_v[sl] = neg
        den_v[sl] = zer
        return carry

    lax.fori_loop(0, _NP // 16, init, 0)
    pltpu.sync_copy(dst_hbm.at[pl.ds(base, ew)], di_all)
    iota = lax.iota(jnp.int32, 16)

    NCH = ew // _CK

    def issue_idx(ci, t):
        b2 = base + ci * _CK
        pltpu.async_copy(src_hbm.at[pl.ds(b2, _CK)], si_l[t], semA[t])
        pltpu.async_copy(dst_hbm.at[pl.ds(b2, _CK)], di_l[t], semA[t])
        pltpu.async_copy(eh_hbm.at[pl.ds(b2, _CK)], e_l[t], semA[t])

    def wait_idx(ci, t):
        b2 = base + ci * _CK
        pltpu.make_async_copy(src_hbm.at[pl.ds(b2, _CK)], si_l[t],
                              semA[t]).wait()
        pltpu.make_async_copy(dst_hbm.at[pl.ds(b2, _CK)], di_l[t],
                              semA[t]).wait()
        pltpu.make_async_copy(eh_hbm.at[pl.ds(b2, _CK)], e_l[t],
                              semA[t]).wait()

    def issue_gather(t):
        pltpu.async_copy(nh_hbm.at[si_l[t]], s_l[t], semB[t])
        pltpu.async_copy(nh_hbm.at[di_l[t]], d_l[t], semB[t])

    def wait_gather(t):
        pltpu.make_async_copy(nh_hbm.at[si_l[t]], s_l[t], semB[t]).wait()
        pltpu.make_async_copy(nh_hbm.at[di_l[t]], d_l[t], semB[t]).wait()

    def compute(ci, t):
        for g in range(_CK // 16):
            ridx = g * 16 + iota

            def cloop(cc, acc):
                cidx = (iota + cc) & 127
                sv = plsc.load_gather(s_l[t], [ridx, cidx])
                ev = plsc.load_gather(e_l[t], [ridx, cidx])
                dv = plsc.load_gather(d_l[t], [ridx, cidx])
                return acc + (sv + ev) * dv

            acc = lax.fori_loop(0, 128, cloop, jnp.zeros((16,), jnp.float32))
            attn_v[pl.ds(ci * _CK + g * 16, 16)] = acc

    def one_iter(ci, t):
        u = 1 - t
        wait_gather(t)
        wait_idx(ci + 1, u)
        issue_gather(u)
        compute(ci, t)
        issue_idx(jnp.minimum(ci + 2, NCH - 1), t)

    issue_idx(0, 0)
    wait_idx(0, 0)
    issue_gather(0)
    issue_idx(1, 1)

    def pair(k, carry):
        one_iter(2 * k, 0)
        one_iter(2 * k + 1, 1)
        return carry

    lax.fori_loop(0, (NCH - 1) // 2, pair, 0)
    wait_gather(0)
    compute(NCH - 1, 0)
    wait_idx(NCH - 1, 1)

    def mloop(j, carry):
        sl = pl.ds(j * 16, 16)
        dd16 = di_all[sl]
        av = attn_v[sl]

        def mcond(pending):
            return jnp.any(pending)

        def mbody(pending):
            cur = plsc.load_gather(m_v, [dd16])
            plsc.store_scatter(m_v, [dd16], jnp.maximum(cur, av),
                               mask=pending)
            cur2 = plsc.load_gather(m_v, [dd16])
            return pending & (cur2 < av)

        lax.while_loop(mcond, mbody, iota < 16)
        return carry

    lax.fori_loop(0, ew // 16, mloop, 0)

    def eloop(j, carry):
        sl = pl.ds(j * 16, 16)
        dd16 = di_all[sl]
        mv = plsc.load_gather(m_v, [dd16])
        ex = jnp.exp(attn_v[sl] - mv)
        plsc.addupdate_scatter(den_v, [dd16], ex)
        return carry

    lax.fori_loop(0, ew // 16, eloop, 0)
    pltpu.sync_copy(attn_v, attn_hbm.at[pl.ds(base, ew)])
    pltpu.sync_copy(m_v, maxp_hbm.at[pl.ds(wid * _NP, _NP)])
    pltpu.sync_copy(den_v, denp_hbm.at[pl.ds(wid * _NP, _NP)])


def _attn_sc(nh, eh, src, dst):
    E = eh.shape[0]
    ew = E // _NW
    mesh = plsc.VectorSubcoreMesh(core_axis_name="c", subcore_axis_name="s")
    f = pl.kernel(
        _attn_body,
        out_type=[
            jax.ShapeDtypeStruct((E,), jnp.float32),
            jax.ShapeDtypeStruct((_NW * _NP,), jnp.float32),
            jax.ShapeDtypeStruct((_NW * _NP,), jnp.float32),
        ],
        mesh=mesh,
        scratch_types=[
            pltpu.VMEM((_CK,), jnp.int32),
            pltpu.VMEM((_CK,), jnp.int32),
            pltpu.VMEM((_CK,), jnp.int32),
            pltpu.VMEM((_CK,), jnp.int32),
            pltpu.VMEM((_CK, 128), jnp.float32),
            pltpu.VMEM((_CK, 128), jnp.float32),
            pltpu.VMEM((_CK, 128), jnp.float32),
            pltpu.VMEM((_CK, 128), jnp.float32),
            pltpu.VMEM((_CK, 128), jnp.float32),
            pltpu.VMEM((_CK, 128), jnp.float32),
            pltpu.VMEM((ew,), jnp.float32),
            pltpu.VMEM((ew,), jnp.int32),
            pltpu.VMEM((_NP,), jnp.float32),
            pltpu.VMEM((_NP,), jnp.float32),
            pltpu.SemaphoreType.DMA,
            pltpu.SemaphoreType.DMA,
            pltpu.SemaphoreType.DMA,
            pltpu.SemaphoreType.DMA,
        ],
        compiler_params=pltpu.CompilerParams(needs_layout_passes=False),
    )
    return f(nh, eh, src, dst)


_NHALF = 5056
_NPH = 5120


def _scatter_body(nh_hbm, attn_hbm, src_hbm, dst_hbm, maxp_hbm, denp_hbm,
                  nzp_hbm, M_hbm, D_hbm,
                  si_0, si_1, di_0, di_1, di2_v, at_0, at_1, a_v, s_0, s_1,
                  zbuf, mw_buf, dw_buf, M_v, D_v, nz_sh,
                  sem, semA0, semA1, semB0, semB1):
    si_l, di_l, at_l, s_l = [si_0, si_1], [di_0, di_1], [at_0, at_1], [s_0, s_1]
    semA = [semA0, semA1]
    semB = [semB0, semB1]
    c = lax.axis_index("c")
    s = lax.axis_index("s")
    wid = s * _NC + c
    E = attn_hbm.shape[0]
    ew = E // _NW
    base = wid * ew
    ns_base = s * _NPS
    iota = lax.iota(jnp.int32, 16)
    zer16 = jnp.zeros((16,), jnp.float32)
    nq = (_NPS + 15) // 16

    def minit(q, carry):
        lidx = q * 16 + iota
        msk = lidx < _NPS
        plsc.store_scatter(M_v, [ns_base + lidx],
                           jnp.full((16,), -1e30, jnp.float32), mask=msk)
        plsc.store_scatter(D_v, [ns_base + lidx], zer16, mask=msk)
        return carry

    lax.fori_loop(0, nq, minit, 0)

    def wmax(w, carry):
        pltpu.sync_copy(maxp_hbm.at[pl.ds(w * _NP + ns_base, _NPS)],
                        mw_buf.at[pl.ds(0, _NPS)])

        def q1(q, carry2):
            lidx = q * 16 + iota
            msk = lidx < _NPS
            idxs = ns_base + lidx
            cur = plsc.load_gather(M_v, [idxs], mask=msk)
            mw = plsc.load_gather(mw_buf, [lidx], mask=msk)
            plsc.store_scatter(M_v, [idxs], jnp.maximum(cur, mw), mask=msk)
            return carry2

        lax.fori_loop(0, nq, q1, 0)
        return carry

    lax.fori_loop(0, _NW, wmax, 0)

    def wden(w, carry):
        pltpu.sync_copy(maxp_hbm.at[pl.ds(w * _NP + ns_base, _NPS)],
                        mw_buf.at[pl.ds(0, _NPS)])
        pltpu.sync_copy(denp_hbm.at[pl.ds(w * _NP + ns_base, _NPS)],
                        dw_buf.at[pl.ds(0, _NPS)])

        def q2(q, carry2):
            lidx = q * 16 + iota
            msk = lidx < _NPS
            idxs = ns_base + lidx
            cur = plsc.load_gather(D_v, [idxs], mask=msk)
            mw = plsc.load_gather(mw_buf, [lidx], mask=msk)
            dw = plsc.load_gather(dw_buf, [lidx], mask=msk)
            mfin = plsc.load_gather(M_v, [idxs], mask=msk)
            plsc.store_scatter(D_v, [idxs], cur + dw * jnp.exp(mw - mfin),
                               mask=msk)
            return carry2

        lax.fori_loop(0, nq, q2, 0)
        return carry

    lax.fori_loop(0, _NW, wden, 0)
    pltpu.sync_copy(M_v.at[pl.ds(ns_base, _NPS)], M_hbm.at[pl.ds(ns_base, _NPS)])
    pltpu.sync_copy(D_v.at[pl.ds(ns_base, _NPS)], D_hbm.at[pl.ds(ns_base, _NPS)])
    plsc.subcore_barrier()
    pltpu.sync_copy(M_hbm, M_v.at[pl.ds(0, _NP)])
    pltpu.sync_copy(D_hbm, D_v.at[pl.ds(0, _NP)])

    def zero_zbuf():
        msk8 = iota < 8

        def zcol(cc, carry2):
            cidx = iota * 0 + cc
            plsc.store_scatter(zbuf, [iota, cidx], zer16, mask=msk8)
            return carry2

        lax.fori_loop(0, 128, zcol, 0)

    ws_base = s * (_NPH // _NS)
    for p in range(2):
        zero_zbuf()
        for q in range(_NPH // _NS // 8):
            pltpu.sync_copy(zbuf, nz_sh.at[pl.ds(ws_base + q * 8, 8)])
        plsc.subcore_barrier()

        NCH = ew // _CK

        def issue_idx(ci, t):
            b2 = base + ci * _CK
            pltpu.async_copy(src_hbm.at[pl.ds(b2, _CK)], si_l[t], semA[t])
            pltpu.async_copy(dst_hbm.at[pl.ds(b2, _CK)], di_l[t], semA[t])
            pltpu.async_copy(attn_hbm.at[pl.ds(b2, _CK)], at_l[t], semA[t])

        def wait_idx(ci, t):
            b2 = base + ci * _CK
            pltpu.make_async_copy(src_hbm.at[pl.ds(b2, _CK)], si_l[t],
                                  semA[t]).wait()
            pltpu.make_async_copy(dst_hbm.at[pl.ds(b2, _CK)], di_l[t],
                                  semA[t]).wait()
            pltpu.make_async_copy(attn_hbm.at[pl.ds(b2, _CK)], at_l[t],
                                  semA[t]).wait()

        def compute_scale(t):
            for g in range(_CK // 16):
                sl = pl.ds(g * 16, 16)
                ridx = g * 16 + iota
                dl = di_l[t][sl]
                mv = plsc.load_gather(M_v, [dl])
                dv = plsc.load_gather(D_v, [dl])
                a16 = jnp.exp(at_l[t][sl] - mv) / dv
                loc = dl - p * _NHALF
                ok = (loc >= 0) & (loc < _NHALF)
                di2_v[sl] = jnp.where(ok, loc, _NHALF)

                def ccol(cc, carry2):
                    cidx = (iota + cc) & 127
                    sv = plsc.load_gather(s_l[t], [ridx, cidx])
                    plsc.store_scatter(s_l[t], [ridx, cidx], sv * a16)
                    return carry2

                lax.fori_loop(0, 128, ccol, 0)

        def one_iter(ci, t):
            u = 1 - t
            pltpu.make_async_copy(nh_hbm.at[si_l[t]], s_l[t],
                                  semB[t]).wait()
            compute_scale(t)
            wait_idx(ci + 1, u)
            pltpu.async_copy(nh_hbm.at[si_l[u]], s_l[u], semB[u])
            issue_idx(jnp.minimum(ci + 2, NCH - 1), t)
            pltpu.sync_copy(s_l[t], nz_sh.at[di2_v], add=True)

        issue_idx(0, 0)
        wait_idx(0, 0)
        pltpu.async_copy(nh_hbm.at[si_l[0]], s_l[0], semB[0])
        issue_idx(1, 1)

        def pair(k, carry):
            one_iter(2 * k, 0)
            one_iter(2 * k + 1, 1)
            return carry

        lax.fori_loop(0, (NCH - 1) // 2, pair, 0)
        pltpu.make_async_copy(nh_hbm.at[si_l[0]], s_l[0], semB[0]).wait()
        compute_scale(0)
        pltpu.sync_copy(s_l[0], nz_sh.at[di2_v], add=True)
        wait_idx(NCH - 1, 1)
        plsc.subcore_barrier()
        for q in range(_NPH // _NS // 8):
            pltpu.sync_copy(nz_sh.at[pl.ds(ws_base + q * 8, 8)], zbuf)
            pltpu.sync_copy(zbuf,
                            nzp_hbm.at[c, p, pl.ds(ws_base + q * 8, 8)])


def _scatter_sc(nh, attn, src, dst, maxp, denp):
    E = attn.shape[0]
    mesh = plsc.VectorSubcoreMesh(core_axis_name="c", subcore_axis_name="s")
    f = pl.kernel(
        _scatter_body,
        out_type=[
            jax.ShapeDtypeStruct((_NC, 2, _NPH, 128), jnp.float32),
            jax.ShapeDtypeStruct((_NP,), jnp.float32),
            jax.ShapeDtypeStruct((_NP,), jnp.float32),
        ],
        mesh=mesh,
        scratch_types=[
            pltpu.VMEM((_CK,), jnp.int32),
            pltpu.VMEM((_CK,), jnp.int32),
            pltpu.VMEM((_CK,), jnp.int32),
            pltpu.VMEM((_CK,), jnp.int32),
            pltpu.VMEM((_CK,), jnp.int32),
            pltpu.VMEM((_CK,), jnp.float32),
            pltpu.VMEM((_CK,), jnp.float32),
            pltpu.VMEM((_CK,), jnp.float32),
            pltpu.VMEM((_CK, 128), jnp.float32),
            pltpu.VMEM((_CK, 128), jnp.float32),
            pltpu.VMEM((8, 128), jnp.float32),
            pltpu.VMEM((640,), jnp.float32),
            pltpu.VMEM((640,), jnp.float32),
            pltpu.VMEM((_NP + 32,), jnp.float32),
            pltpu.VMEM((_NP + 32,), jnp.float32),
            pltpu.VMEM_SHARED((_NPH, 128), jnp.float32),
            pltpu.SemaphoreType.DMA,
            pltpu.SemaphoreType.DMA,
            pltpu.SemaphoreType.DMA,
            pltpu.SemaphoreType.DMA,
            pltpu.SemaphoreType.DMA,
        ],
        compiler_params=pltpu.CompilerParams(needs_layout_passes=False),
    )
    return f(nh, attn, src, dst, maxp, denp)[0]


def _epre_body(epsp_hbm, eh_hbm, nz_hbm, src_hbm, dst_hbm, out_hbm,
               epsp_v, eh_0, eh_1, s_0, s_1, d_0, d_1, si_0, si_1,
               di_0, di_1, semA0, semA1, semB0, semB1):
    eh_l, s_l, d_l = [eh_0, eh_1], [s_0, s_1], [d_0, d_1]
    si_l, di_l = [si_0, si_1], [di_0, di_1]
    semA = [semA0, semA1]
    semB = [semB0, semB1]
    c = lax.axis_index("c")
    s = lax.axis_index("s")
    wid = s * _NC + c
    E = eh_hbm.shape[0]
    ew = E // _NW
    base = wid * ew
    NCH = ew // _CK
    pltpu.sync_copy(epsp_hbm, epsp_v)
    eps_sl = [epsp_v[pl.ds(t * 16, 16)] for t in range(8)]

    def issue_idx(ci, t):
        b2 = base + ci * _CK
        pltpu.async_copy(src_hbm.at[pl.ds(b2, _CK)], si_l[t], semA[t])
        pltpu.async_copy(dst_hbm.at[pl.ds(b2, _CK)], di_l[t], semA[t])
        pltpu.async_copy(eh_hbm.at[pl.ds(b2, _CK)], eh_l[t], semA[t])

    def wait_idx(ci, t):
        b2 = base + ci * _CK
        pltpu.make_async_copy(src_hbm.at[pl.ds(b2, _CK)], si_l[t],
                              semA[t]).wait()
        pltpu.make_async_copy(dst_hbm.at[pl.ds(b2, _CK)], di_l[t],
                              semA[t]).wait()
        pltpu.make_async_copy(eh_hbm.at[pl.ds(b2, _CK)], eh_l[t],
                              semA[t]).wait()

    def issue_gather(t):
        pltpu.async_copy(nz_hbm.at[si_l[t]], s_l[t], semB[t])
        pltpu.async_copy(nz_hbm.at[di_l[t]], d_l[t], semB[t])

    def wait_gather(t):
        pltpu.make_async_copy(nz_hbm.at[si_l[t]], s_l[t], semB[t]).wait()
        pltpu.make_async_copy(nz_hbm.at[di_l[t]], d_l[t], semB[t]).wait()

    def compute_store(ci, t):
        def row(j, carry2):
            for tt in range(8):
                sl = pl.ds(tt * 16, 16)
                eh_l[t][j, sl] = (eh_l[t][j, sl] * eps_sl[tt]
                                  + s_l[t][j, sl] - d_l[t][j, sl])
            return carry2

        lax.fori_loop(0, _CK, row, 0)
        pltpu.sync_copy(eh_l[t], out_hbm.at[pl.ds(base + ci * _CK, _CK)])

    def one_iter(ci, t):
        u = 1 - t
        wait_gather(t)
        wait_idx(ci + 1, u)
        issue_gather(u)
        compute_store(ci, t)
        issue_idx(jnp.minimum(ci + 2, NCH - 1), t)

    issue_idx(0, 0)
    wait_idx(0, 0)
    issue_gather(0)
    issue_idx(1, 1)

    def pair(k, carry):
        one_iter(2 * k, 0)
        one_iter(2 * k + 1, 1)
        return carry

    lax.fori_loop(0, (NCH - 1) // 2, pair, 0)
    wait_gather(0)
    compute_store(NCH - 1, 0)
    wait_idx(NCH - 1, 1)


def _epre_sc(epsp, eh, nz, src, dst):
    E = eh.shape[0]
    mesh = plsc.VectorSubcoreMesh(core_axis_name="c", subcore_axis_name="s")
    f = pl.kernel(
        _epre_body,
        out_type=jax.ShapeDtypeStruct((E, 128), jnp.float32),
        mesh=mesh,
        scratch_types=[
            pltpu.VMEM((128,), jnp.float32),
            pltpu.VMEM((_CK, 128), jnp.float32),
            pltpu.VMEM((_CK, 128), jnp.float32),
            pltpu.VMEM((_CK, 128), jnp.float32),
            pltpu.VMEM((_CK, 128), jnp.float32),
            pltpu.VMEM((_CK, 128), jnp.float32),
            pltpu.VMEM((_CK, 128), jnp.float32),
            pltpu.VMEM((_CK,), jnp.int32),
            pltpu.VMEM((_CK,), jnp.int32),
            pltpu.VMEM((_CK,), jnp.int32),
            pltpu.VMEM((_CK,), jnp.int32),
            pltpu.SemaphoreType.DMA,
            pltpu.SemaphoreType.DMA,
            pltpu.SemaphoreType.DMA,
            pltpu.SemaphoreType.DMA,
        ],
    )
    return f(epsp, eh, nz, src, dst)


def _mlp_stats_body(x_ref, w1_ref, b1_ref, w2_ref, b2_ref,
                    y_ref, s1_ref, s2_ref):
    i = pl.program_id(0)
    x = x_ref[...]
    h = jnp.maximum(
        jnp.dot(x, w1_ref[...], preferred_element_type=jnp.float32)
        + b1_ref[...], 0.0)
    y = (jnp.dot(h, w2_ref[...], preferred_element_type=jnp.float32)
         + b2_ref[...])
    y_ref[...] = y

    @pl.when(i == 0)
    def _init():
        s1_ref[...] = jnp.zeros_like(s1_ref)
        s2_ref[...] = jnp.zeros_like(s2_ref)

    s1_ref[...] += jnp.sum(y, axis=0, keepdims=True)
    s2_ref[...] += jnp.sum(y * y, axis=0, keepdims=True)


def _mlp_stats(x, w1, b1, w2, b2, block):
    n = x.shape[0]
    grid = n // block
    y, s1, s2 = pl.pallas_call(
        _mlp_stats_body,
        grid=(grid,),
        in_specs=[
            pl.BlockSpec((block, 128), lambda i: (i, 0)),
            pl.BlockSpec((128, 128), lambda i: (0, 0)),
            pl.BlockSpec((1, 128), lambda i: (0, 0)),
            pl.BlockSpec((128, 128), lambda i: (0, 0)),
            pl.BlockSpec((1, 128), lambda i: (0, 0)),
        ],
        out_specs=[
            pl.BlockSpec((block, 128), lambda i: (i, 0)),
            pl.BlockSpec((1, 128), lambda i: (0, 0)),
            pl.BlockSpec((1, 128), lambda i: (0, 0)),
        ],
        out_shape=[
            jax.ShapeDtypeStruct((n, 128), jnp.float32),
            jax.ShapeDtypeStruct((1, 128), jnp.float32),
            jax.ShapeDtypeStruct((1, 128), jnp.float32),
        ],
    )(x, w1, b1.reshape(1, 128), w2, b2.reshape(1, 128))
    return y, s1, s2


def _bn_body(y_ref, mu_ref, rstd_ref, gamma_ref, beta_ref, o_ref):
    o_ref[...] = ((y_ref[...] - mu_ref[...]) * rstd_ref[...]
                  * gamma_ref[...] + beta_ref[...])


def _bn_apply(y, mu, rstd, gamma, beta, block):
    n = y.shape[0]
    return pl.pallas_call(
        _bn_body,
        grid=(n // block,),
        in_specs=[
            pl.BlockSpec((block, 128), lambda i: (i, 0)),
            pl.BlockSpec((1, 128), lambda i: (0, 0)),
            pl.BlockSpec((1, 128), lambda i: (0, 0)),
            pl.BlockSpec((1, 128), lambda i: (0, 0)),
            pl.BlockSpec((1, 128), lambda i: (0, 0)),
        ],
        out_specs=pl.BlockSpec((block, 128), lambda i: (i, 0)),
        out_shape=jax.ShapeDtypeStruct((n, 128), jnp.float32),
    )(y, mu, rstd, gamma.reshape(1, 128), beta.reshape(1, 128))


def _mlp_bn(x, w1, b1, w2, b2, gamma, beta, block):
    n = x.shape[0]
    y, s1, s2 = _mlp_stats(x, w1, b1, w2, b2, block)
    mu = s1 / n
    var = s2 / n - mu * mu
    rstd = jax.lax.rsqrt(var + 1e-5)
    return _bn_apply(y, mu, rstd, gamma, beta, block)


def kernel(nh, eh, edge_index,
           nf_W1, nf_b1, nf_W2, nf_b2, nf_eps, nf_gamma, nf_beta,
           ef_W1, ef_b1, ef_W2, ef_b2, ef_eps, ef_gamma, ef_beta):
    N = nh.shape[0]
    src, dst = edge_index[0], edge_index[1]
    attn, maxp, denp = _attn_sc(nh, eh, src, dst)
    nzp = _scatter_sc(nh, attn, src, dst, maxp, denp)
    nz = jnp.concatenate([(nzp[0, 0] + nzp[1, 0])[:_NHALF],
                          (nzp[0, 1] + nzp[1, 1])[:N - _NHALF]], axis=0)
    n_pre = (1.0 + nf_eps) * nh + nz
    e_pre = _epre_sc(1.0 + ef_eps, eh, nz, src, dst)
    n_h = _mlp_bn(n_pre, nf_W1, nf_b1, nf_W2, nf_b2, nf_gamma, nf_beta, 1000)
    e_h = _mlp_bn(e_pre, ef_W1, ef_b1, ef_W2, ef_b2, ef_gamma, ef_beta, 1000)
    return (n_h, e_h)
